# baseline (speedup 1.0000x reference)
import functools

import jax
import jax.numpy as jnp
from jax.experimental import pallas as pl
from jax.experimental.pallas import tpu as pltpu


def _proj_body(x_ref, wn_ref, wr_ref, p_ref, r_ref):
    x = x_ref[...]
    p_ref[...] = jnp.dot(x, wn_ref[...], preferred_element_type=jnp.float32)
    r_ref[...] = jnp.dot(x, wr_ref[...], preferred_element_type=jnp.float32)


def _ln_from_parts(agg, r, cnt, bn, g, b):
    inv = 1.0 / jnp.maximum(cnt[:, 0:1], 1.0)
    z = agg * inv + bn + r
    mu = jnp.mean(z, axis=-1, keepdims=True)
    var = jnp.mean((z - mu) ** 2, axis=-1, keepdims=True)
    h = (z - mu) * jax.lax.rsqrt(var + 1e-5) * g + b
    return jnp.maximum(h, 0.0)


def _fused_ln_proj_body(agg_ref, r_ref, cnt_ref, bn_ref, g_ref, b_ref,
                        wn_ref, wr_ref, p1_ref, r1_ref):
    h = _ln_from_parts(agg_ref[0], r_ref[0], cnt_ref[...],
                       bn_ref[...], g_ref[...], b_ref[...])
    p1_ref[0] = jnp.dot(h, wn_ref[...], preferred_element_type=jnp.float32)
    r1_ref[0] = jnp.dot(h, wr_ref[...], preferred_element_type=jnp.float32)


def _ln_mean_body(agg_ref, r_ref, cnt_ref, bn_ref, g_ref, b_ref, hsum_ref):
    nb = pl.program_id(1)
    h = _ln_from_parts(agg_ref[0], r_ref[0], cnt_ref[...],
                       bn_ref[...], g_ref[...], b_ref[...])

    @pl.when(nb == 0)
    def _():
        hsum_ref[...] = jnp.zeros_like(hsum_ref)

    s = jnp.sum(h, axis=0)
    hsum_ref[...] += jnp.broadcast_to(s[None, None, :], hsum_ref.shape)


def _gru_body(hseq_ref, wih_ref, whh_ref, bih_ref, bhh_ref, hw_ref, hb_ref,
              y_ref, *, n_nodes):
    B, T, _ = hseq_ref.shape
    Ht = whh_ref.shape[0]
    h = jnp.zeros((B, Ht), jnp.float32)
    scale = 1.0 / float(n_nodes)
    for t in range(T):
        ht = hseq_ref[:, t, :] * scale
        gi = jnp.dot(ht, wih_ref[...], preferred_element_type=jnp.float32) + bih_ref[...]
        gh = jnp.dot(h, whh_ref[...], preferred_element_type=jnp.float32) + bhh_ref[...]
        ir, iz, inn = gi[:, :Ht], gi[:, Ht:2 * Ht], gi[:, 2 * Ht:]
        hr, hz, hn = gh[:, :Ht], gh[:, Ht:2 * Ht], gh[:, 2 * Ht:]
        r = jax.nn.sigmoid(ir + hr)
        z = jax.nn.sigmoid(iz + hz)
        n = jnp.tanh(inn + r * hn)
        h = (1.0 - z) * n + z * h
    y_ref[...] = jnp.dot(h, hw_ref[...], preferred_element_type=jnp.float32) + hb_ref[...]


def _project(x, wn, wr, mb):
    M, F = x.shape
    Hg = wn.shape[1]
    return pl.pallas_call(
        _proj_body,
        grid=(M // mb,),
        in_specs=[
            pl.BlockSpec((mb, F), lambda i: (i, 0)),
            pl.BlockSpec((F, Hg), lambda i: (0, 0)),
            pl.BlockSpec((F, Hg), lambda i: (0, 0)),
        ],
        out_specs=[
            pl.BlockSpec((mb, Hg), lambda i: (i, 0)),
            pl.BlockSpec((mb, Hg), lambda i: (i, 0)),
        ],
        out_shape=[
            jax.ShapeDtypeStruct((M, Hg), jnp.float32),
            jax.ShapeDtypeStruct((M, Hg), jnp.float32),
        ],
    )(x, wn, wr)


def _fused_ln_proj(agg, r, cntb, bn, g, b, wn, wr, nb):
    BT, N, Hg = agg.shape
    grid = (BT, N // nb)
    row = pl.BlockSpec((1, nb, Hg), lambda i, j: (i, j, 0))
    vec = pl.BlockSpec((1, Hg), lambda i, j: (0, 0))
    mat = pl.BlockSpec((Hg, Hg), lambda i, j: (0, 0))
    cnt = pl.BlockSpec((nb, cntb.shape[1]), lambda i, j: (j, 0))
    return pl.pallas_call(
        _fused_ln_proj_body,
        grid=grid,
        in_specs=[row, row, cnt, vec, vec, vec, mat, mat],
        out_specs=[row, row],
        out_shape=[
            jax.ShapeDtypeStruct((BT, N, Hg), jnp.float32),
            jax.ShapeDtypeStruct((BT, N, Hg), jnp.float32),
        ],
    )(agg, r, cntb, bn, g, b, wn, wr)


def _ln_mean(agg, r, cntb, bn, g, b, nb):
    BT, N, Hg = agg.shape
    grid = (BT, N // nb)
    row = pl.BlockSpec((1, nb, Hg), lambda i, j: (i, j, 0))
    vec = pl.BlockSpec((1, Hg), lambda i, j: (0, 0))
    cnt = pl.BlockSpec((nb, cntb.shape[1]), lambda i, j: (j, 0))
    return pl.pallas_call(
        _ln_mean_body,
        grid=grid,
        in_specs=[row, row, cnt, vec, vec, vec],
        out_specs=pl.BlockSpec((1, 8, Hg), lambda i, j: (i, 0, 0)),
        out_shape=jax.ShapeDtypeStruct((BT, 8, Hg), jnp.float32),
    )(agg, r, cntb, bn, g, b)[:, 0, :]


def _gru_head(hseq, wih_t, whh_t, bih, bhh, head_w, head_b, n_nodes):
    B, T, Hg = hseq.shape
    Ht = whh_t.shape[0]
    full = lambda s: pl.BlockSpec(s, lambda: tuple(0 for _ in s))
    return pl.pallas_call(
        functools.partial(_gru_body, n_nodes=n_nodes),
        in_specs=[full((B, T, Hg)), full((Hg, 3 * Ht)), full((Ht, 3 * Ht)),
                  full((1, 3 * Ht)), full((1, 3 * Ht)), full((Ht, 1)),
                  full((1, 1))],
        out_specs=full((B, 1)),
        out_shape=jax.ShapeDtypeStruct((B, 1), jnp.float32),
    )(hseq, wih_t, whh_t, bih, bhh, head_w, head_b)


def _segment_mean_sums(y, src, dst, n):
    def one(yb):
        return jax.ops.segment_sum(yb[src, :], dst, num_segments=n)
    return jax.vmap(one)(y)


def kernel(x_seq, edge_index, W_neigh0, b_neigh0, W_root0, ln_g0, ln_b0,
           W_neigh1, b_neigh1, W_root1, ln_g1, ln_b1,
           gru_Wih, gru_Whh, gru_bih, gru_bhh, head_W, head_b):
    B, T, N, F = x_seq.shape
    E = edge_index.shape[1]
    Hg = W_neigh0.shape[1]
    Ht = gru_Whh.shape[1]
    BT = B * T
    M = BT * N

    src = edge_index[0].astype(jnp.int32)
    dst = edge_index[1].astype(jnp.int32)

    x = x_seq.reshape(M, F)
    P0, R0 = _project(x, W_neigh0, W_root0, mb=2000)

    cnt = jax.ops.segment_sum(jnp.ones((E,), jnp.float32), dst, num_segments=N)
    cntb = jnp.broadcast_to(cnt[:, None], (N, 16))

    agg0 = _segment_mean_sums(P0.reshape(BT, N, Hg), src, dst, N)

    P1, R1 = _fused_ln_proj(agg0, R0.reshape(BT, N, Hg), cntb,
                            b_neigh0.reshape(1, Hg), ln_g0.reshape(1, Hg),
                            ln_b0.reshape(1, Hg), W_neigh1, W_root1, nb=1000)

    agg1 = _segment_mean_sums(P1, src, dst, N)

    hsum = _ln_mean(agg1, R1, cntb, b_neigh1.reshape(1, Hg),
                    ln_g1.reshape(1, Hg), ln_b1.reshape(1, Hg), nb=1000)

    y = _gru_head(hsum.reshape(B, T, Hg), gru_Wih.T, gru_Whh.T,
                  gru_bih.reshape(1, 3 * Ht), gru_bhh.reshape(1, 3 * Ht),
                  head_W, head_b.reshape(1, 1), n_nodes=N)
    return y[:, 0]


# speedup vs baseline: 12.8484x; 12.8484x over previous
import functools

import jax
import jax.numpy as jnp
from jax import lax
from jax.experimental import pallas as pl
from jax.experimental.pallas import tpu as pltpu
from jax.experimental.pallas import tpu_sc as plsc


def _proj_body(x_ref, wn_ref, wr_ref, p_ref, r_ref):
    x = x_ref[...]
    p_ref[...] = jnp.dot(x, wn_ref[...], preferred_element_type=jnp.float32)
    r_ref[...] = jnp.dot(x, wr_ref[...], preferred_element_type=jnp.float32)


def _ln_from_parts(agg, r, cnt, bn, g, b):
    inv = 1.0 / jnp.maximum(cnt[:, 0:1], 1.0)
    z = agg * inv + bn + r
    mu = jnp.mean(z, axis=-1, keepdims=True)
    var = jnp.mean((z - mu) ** 2, axis=-1, keepdims=True)
    h = (z - mu) * lax.rsqrt(var + 1e-5) * g + b
    return jnp.maximum(h, 0.0)


def _fused_ln_proj_body(agg_ref, r_ref, cnt_ref, bn_ref, g_ref, b_ref,
                        wn_ref, wr_ref, p1_ref, r1_ref):
    h = _ln_from_parts(agg_ref[0], r_ref[0], cnt_ref[...],
                       bn_ref[...], g_ref[...], b_ref[...])
    p1_ref[0] = jnp.dot(h, wn_ref[...], preferred_element_type=jnp.float32)
    r1_ref[0] = jnp.dot(h, wr_ref[...], preferred_element_type=jnp.float32)


def _ln_mean_body(agg_ref, r_ref, cnt_ref, bn_ref, g_ref, b_ref, hsum_ref):
    nb = pl.program_id(1)
    h = _ln_from_parts(agg_ref[0], r_ref[0], cnt_ref[...],
                       bn_ref[...], g_ref[...], b_ref[...])

    @pl.when(nb == 0)
    def _():
        hsum_ref[...] = jnp.zeros_like(hsum_ref)

    s = jnp.sum(h, axis=0)
    hsum_ref[...] += jnp.broadcast_to(s[None, None, :], hsum_ref.shape)


def _gru_body(hseq_ref, wih_ref, whh_ref, bih_ref, bhh_ref, hw_ref, hb_ref,
              y_ref, *, n_nodes):
    B, T, _ = hseq_ref.shape
    Ht = whh_ref.shape[0]
    h = jnp.zeros((B, Ht), jnp.float32)
    scale = 1.0 / float(n_nodes)
    for t in range(T):
        ht = hseq_ref[:, t, :] * scale
        gi = jnp.dot(ht, wih_ref[...], preferred_element_type=jnp.float32) + bih_ref[...]
        gh = jnp.dot(h, whh_ref[...], preferred_element_type=jnp.float32) + bhh_ref[...]
        ir, iz, inn = gi[:, :Ht], gi[:, Ht:2 * Ht], gi[:, 2 * Ht:]
        hr, hz, hn = gh[:, :Ht], gh[:, Ht:2 * Ht], gh[:, 2 * Ht:]
        r = jax.nn.sigmoid(ir + hr)
        z = jax.nn.sigmoid(iz + hz)
        n = jnp.tanh(inn + r * hn)
        h = (1.0 - z) * n + z * h
    y_ref[...] = jnp.dot(h, hw_ref[...], preferred_element_type=jnp.float32) + hb_ref[...]


def _project(x, wn, wr, mb):
    M, F = x.shape
    Hg = wn.shape[1]
    return pl.pallas_call(
        _proj_body,
        grid=(M // mb,),
        in_specs=[
            pl.BlockSpec((mb, F), lambda i: (i, 0)),
            pl.BlockSpec((F, Hg), lambda i: (0, 0)),
            pl.BlockSpec((F, Hg), lambda i: (0, 0)),
        ],
        out_specs=[
            pl.BlockSpec((mb, Hg), lambda i: (i, 0)),
            pl.BlockSpec((mb, Hg), lambda i: (i, 0)),
        ],
        out_shape=[
            jax.ShapeDtypeStruct((M, Hg), jnp.float32),
            jax.ShapeDtypeStruct((M, Hg), jnp.float32),
        ],
    )(x, wn, wr)


def _fused_ln_proj(agg, r, cntb, bn, g, b, wn, wr, nb):
    BT, N, Hg = agg.shape
    grid = (BT, N // nb)
    row = pl.BlockSpec((1, nb, Hg), lambda i, j: (i, j, 0))
    vec = pl.BlockSpec((1, Hg), lambda i, j: (0, 0))
    mat = pl.BlockSpec((Hg, Hg), lambda i, j: (0, 0))
    cnt = pl.BlockSpec((nb, cntb.shape[1]), lambda i, j: (j, 0))
    return pl.pallas_call(
        _fused_ln_proj_body,
        grid=grid,
        in_specs=[row, row, cnt, vec, vec, vec, mat, mat],
        out_specs=[row, row],
        out_shape=[
            jax.ShapeDtypeStruct((BT, N, Hg), jnp.float32),
            jax.ShapeDtypeStruct((BT, N, Hg), jnp.float32),
        ],
    )(agg, r, cntb, bn, g, b, wn, wr)


def _ln_mean(agg, r, cntb, bn, g, b, nb):
    BT, N, Hg = agg.shape
    grid = (BT, N // nb)
    row = pl.BlockSpec((1, nb, Hg), lambda i, j: (i, j, 0))
    vec = pl.BlockSpec((1, Hg), lambda i, j: (0, 0))
    cnt = pl.BlockSpec((nb, cntb.shape[1]), lambda i, j: (j, 0))
    return pl.pallas_call(
        _ln_mean_body,
        grid=grid,
        in_specs=[row, row, cnt, vec, vec, vec],
        out_specs=pl.BlockSpec((1, 8, Hg), lambda i, j: (i, 0, 0)),
        out_shape=jax.ShapeDtypeStruct((BT, 8, Hg), jnp.float32),
    )(agg, r, cntb, bn, g, b)[:, 0, :]


def _gru_head(hseq, wih_t, whh_t, bih, bhh, head_w, head_b, n_nodes):
    B, T, Hg = hseq.shape
    Ht = whh_t.shape[0]
    full = lambda s: pl.BlockSpec(s, lambda: tuple(0 for _ in s))
    return pl.pallas_call(
        functools.partial(_gru_body, n_nodes=n_nodes),
        in_specs=[full((B, T, Hg)), full((Hg, 3 * Ht)), full((Ht, 3 * Ht)),
                  full((1, 3 * Ht)), full((1, 3 * Ht)), full((Ht, 1)),
                  full((1, 1))],
        out_specs=full((B, 1)),
        out_shape=jax.ShapeDtypeStruct((B, 1), jnp.float32),
    )(hseq, wih_t, whh_t, bih, bhh, head_w, head_b)


_NTILES = 16
_CH = 100
_RPT = 640


def _fill(ref, value):
    rows, cols = ref.shape

    def body(i, _):
        v = jnp.full((16,), value, jnp.float32)
        for j in range(cols // 16):
            ref[i, pl.ds(j * 16, 16)] = v
        return 0

    lax.fori_loop(0, rows, body, 0)


def _tile_slices(s, n):
    row0 = pl.multiple_of(s * _RPT, 8)
    last = n - _RPT * (_NTILES - 1)
    return row0, last


def _zero_slice(zb_v, dst_ref, row0):
    zrows = zb_v.shape[0]
    for j in range(_RPT // zrows):
        pltpu.sync_copy(zb_v, dst_ref.at[pl.ds(row0 + j * zrows, zrows)])


def _write_slice(s, n, acc_ref, hbm_ref):
    row0, last = _tile_slices(s, n)

    @pl.when(s < _NTILES - 1)
    def _():
        pltpu.sync_copy(acc_ref.at[pl.ds(row0, _RPT)],
                        hbm_ref.at[pl.ds(row0, _RPT)])

    @pl.when(s == _NTILES - 1)
    def _():
        base = _RPT * (_NTILES - 1)
        pltpu.sync_copy(acc_ref.at[pl.ds(base, last)],
                        hbm_ref.at[pl.ds(base, last)])


def _agg_body(y_hbm, src3_hbm, dst3_hbm, out_hbm, cnt_hbm, src_v, dst_v,
              rows_v, zb_v, ones_v, acc, sem0, sem1):
    BT, N, W = y_hbm.shape
    nch = src_v.shape[0]
    c = lax.axis_index("c")
    s = lax.axis_index("s")
    row0, _ = _tile_slices(s, N)

    pltpu.sync_copy(src3_hbm.at[s], src_v)
    pltpu.sync_copy(dst3_hbm.at[s], dst_v)
    _fill(zb_v, 0.0)

    @pl.when(c == 0)
    def _():
        _fill(ones_v, 1.0)
        _zero_slice(zb_v, acc, row0)
        plsc.subcore_barrier()

        def cbody(ch, _):
            pltpu.sync_copy(ones_v, acc.at[dst_v.at[ch]], add=True)
            return 0

        lax.fori_loop(0, nch, cbody, 0)
        plsc.subcore_barrier()
        _write_slice(s, N, acc, cnt_hbm)

    npass = BT // 2

    def pass_body(p, _):
        bt = c * npass + p
        _zero_slice(zb_v, acc, row0)
        plsc.subcore_barrier()
        ysrc = y_hbm.at[bt]

        def chunk2(i, _):
            ch0 = i * 2
            ch1 = ch0 + 1
            g0 = pltpu.async_copy(ysrc.at[src_v.at[ch0]], rows_v.at[0], sem0)
            g1 = pltpu.async_copy(ysrc.at[src_v.at[ch1]], rows_v.at[1], sem1)
            g0.wait()
            pltpu.sync_copy(rows_v.at[0], acc.at[dst_v.at[ch0]], add=True)
            g1.wait()
            pltpu.sync_copy(rows_v.at[1], acc.at[dst_v.at[ch1]], add=True)
            return 0

        lax.fori_loop(0, nch // 2, chunk2, 0)
        plsc.subcore_barrier()
        _write_slice(s, N, acc, out_hbm.at[bt])
        return 0

    lax.fori_loop(0, npass, pass_body, 0)


@functools.lru_cache(maxsize=None)
def _make_sc_aggregate(bt, n, w, nch):
    mesh = plsc.VectorSubcoreMesh(core_axis_name="c", subcore_axis_name="s")
    n_pad = _RPT * _NTILES
    return functools.partial(
        pl.kernel,
        out_type=[jax.ShapeDtypeStruct((bt, n, w), jnp.float32),
                  jax.ShapeDtypeStruct((n, w), jnp.float32)],
        scratch_types=[
            pltpu.VMEM((nch, _CH), jnp.int32),
            pltpu.VMEM((nch, _CH), jnp.int32),
            pltpu.VMEM((2, _CH, w), jnp.float32),
            pltpu.VMEM((80, w), jnp.float32),
            pltpu.VMEM((_CH, w), jnp.float32),
            pltpu.VMEM_SHARED((n_pad, w), jnp.float32),
            pltpu.SemaphoreType.DMA,
            pltpu.SemaphoreType.DMA,
        ],
        mesh=mesh,
        compiler_params=pltpu.CompilerParams(use_tc_tiling_on_sc=False),
    )(_agg_body)


def _sc_aggregate(y, src3, dst3):
    BT, N, W = y.shape
    return _make_sc_aggregate(BT, N, W, src3.shape[1])(y, src3, dst3)


def kernel(x_seq, edge_index, W_neigh0, b_neigh0, W_root0, ln_g0, ln_b0,
           W_neigh1, b_neigh1, W_root1, ln_g1, ln_b1,
           gru_Wih, gru_Whh, gru_bih, gru_bhh, head_W, head_b):
    B, T, N, F = x_seq.shape
    E = edge_index.shape[1]
    Hg = W_neigh0.shape[1]
    Ht = gru_Whh.shape[1]
    BT = B * T
    M = BT * N

    ept = E // _NTILES
    src3 = edge_index[0].astype(jnp.int32).reshape(_NTILES, ept // _CH, _CH)
    dst3 = edge_index[1].astype(jnp.int32).reshape(_NTILES, ept // _CH, _CH)

    x = x_seq.reshape(M, F)
    P0, R0 = _project(x, W_neigh0, W_root0, mb=2000)

    agg0, cntb = _sc_aggregate(P0.reshape(BT, N, Hg), src3, dst3)

    P1, R1 = _fused_ln_proj(agg0, R0.reshape(BT, N, Hg), cntb,
                            b_neigh0.reshape(1, Hg), ln_g0.reshape(1, Hg),
                            ln_b0.reshape(1, Hg), W_neigh1, W_root1, nb=1000)

    agg1, _cnt_unused = _sc_aggregate(P1, src3, dst3)

    hsum = _ln_mean(agg1, R1, cntb, b_neigh1.reshape(1, Hg),
                    ln_g1.reshape(1, Hg), ln_b1.reshape(1, Hg), nb=1000)

    y = _gru_head(hsum.reshape(B, T, Hg), gru_Wih.T, gru_Whh.T,
                  gru_bih.reshape(1, 3 * Ht), gru_bhh.reshape(1, 3 * Ht),
                  head_W, head_b.reshape(1, 1), n_nodes=N)
    return y[:, 0]


# speedup vs baseline: 20.8774x; 1.6249x over previous
import functools

import jax
import jax.numpy as jnp
from jax import lax
from jax.experimental import pallas as pl
from jax.experimental.pallas import tpu as pltpu
from jax.experimental.pallas import tpu_sc as plsc


def _proj_body(x_ref, wn_ref, wr_ref, p_ref, r_ref):
    x = x_ref[...]
    p_ref[...] = jnp.dot(x, wn_ref[...], preferred_element_type=jnp.float32)
    r_ref[...] = jnp.dot(x, wr_ref[...], preferred_element_type=jnp.float32)


def _ln_from_parts(agg, r, cnt, bn, g, b):
    inv = 1.0 / jnp.maximum(cnt[:, 0:1], 1.0)
    z = agg * inv + bn + r
    mu = jnp.mean(z, axis=-1, keepdims=True)
    var = jnp.mean((z - mu) ** 2, axis=-1, keepdims=True)
    h = (z - mu) * lax.rsqrt(var + 1e-5) * g + b
    return jnp.maximum(h, 0.0)


def _fused_ln_proj_body(agg_ref, r_ref, cnt_ref, bn_ref, g_ref, b_ref,
                        wn_ref, wr_ref, p1_ref, r1_ref):
    h = _ln_from_parts(agg_ref[0], r_ref[0], cnt_ref[...],
                       bn_ref[...], g_ref[...], b_ref[...])
    p1_ref[0] = jnp.dot(h, wn_ref[...], preferred_element_type=jnp.float32)
    r1_ref[0] = jnp.dot(h, wr_ref[...], preferred_element_type=jnp.float32)


def _ln_mean_body(agg_ref, r_ref, cnt_ref, bn_ref, g_ref, b_ref, hsum_ref):
    nb = pl.program_id(1)
    h = _ln_from_parts(agg_ref[0], r_ref[0], cnt_ref[...],
                       bn_ref[...], g_ref[...], b_ref[...])

    @pl.when(nb == 0)
    def _():
        hsum_ref[...] = jnp.zeros_like(hsum_ref)

    s = jnp.sum(h, axis=0)
    hsum_ref[...] += jnp.broadcast_to(s[None, None, :], hsum_ref.shape)


def _gru_body(hseq_ref, wih_ref, whh_ref, bih_ref, bhh_ref, hw_ref, hb_ref,
              y_ref, *, n_nodes):
    B, T, _ = hseq_ref.shape
    Ht = whh_ref.shape[0]
    h = jnp.zeros((B, Ht), jnp.float32)
    scale = 1.0 / float(n_nodes)
    for t in range(T):
        ht = hseq_ref[:, t, :] * scale
        gi = jnp.dot(ht, wih_ref[...], preferred_element_type=jnp.float32) + bih_ref[...]
        gh = jnp.dot(h, whh_ref[...], preferred_element_type=jnp.float32) + bhh_ref[...]
        ir, iz, inn = gi[:, :Ht], gi[:, Ht:2 * Ht], gi[:, 2 * Ht:]
        hr, hz, hn = gh[:, :Ht], gh[:, Ht:2 * Ht], gh[:, 2 * Ht:]
        r = jax.nn.sigmoid(ir + hr)
        z = jax.nn.sigmoid(iz + hz)
        n = jnp.tanh(inn + r * hn)
        h = (1.0 - z) * n + z * h
    y_ref[...] = jnp.dot(h, hw_ref[...], preferred_element_type=jnp.float32) + hb_ref[...]


def _project(x, wn, wr, mb):
    M, F = x.shape
    Hg = wn.shape[1]
    return pl.pallas_call(
        _proj_body,
        grid=(M // mb,),
        in_specs=[
            pl.BlockSpec((mb, F), lambda i: (i, 0)),
            pl.BlockSpec((F, Hg), lambda i: (0, 0)),
            pl.BlockSpec((F, Hg), lambda i: (0, 0)),
        ],
        out_specs=[
            pl.BlockSpec((mb, Hg), lambda i: (i, 0)),
            pl.BlockSpec((mb, Hg), lambda i: (i, 0)),
        ],
        out_shape=[
            jax.ShapeDtypeStruct((M, Hg), jnp.float32),
            jax.ShapeDtypeStruct((M, Hg), jnp.float32),
        ],
    )(x, wn, wr)


def _fused_ln_proj(agg, r, cntb, bn, g, b, wn, wr, nb):
    BT, N, Hg = agg.shape
    grid = (BT, N // nb)
    row = pl.BlockSpec((1, nb, Hg), lambda i, j: (i, j, 0))
    vec = pl.BlockSpec((1, Hg), lambda i, j: (0, 0))
    mat = pl.BlockSpec((Hg, Hg), lambda i, j: (0, 0))
    cnt = pl.BlockSpec((nb, cntb.shape[1]), lambda i, j: (j, 0))
    return pl.pallas_call(
        _fused_ln_proj_body,
        grid=grid,
        in_specs=[row, row, cnt, vec, vec, vec, mat, mat],
        out_specs=[row, row],
        out_shape=[
            jax.ShapeDtypeStruct((BT, N, Hg), jnp.float32),
            jax.ShapeDtypeStruct((BT, N, Hg), jnp.float32),
        ],
    )(agg, r, cntb, bn, g, b, wn, wr)


def _ln_mean(agg, r, cntb, bn, g, b, nb):
    BT, N, Hg = agg.shape
    grid = (BT, N // nb)
    row = pl.BlockSpec((1, nb, Hg), lambda i, j: (i, j, 0))
    vec = pl.BlockSpec((1, Hg), lambda i, j: (0, 0))
    cnt = pl.BlockSpec((nb, cntb.shape[1]), lambda i, j: (j, 0))
    return pl.pallas_call(
        _ln_mean_body,
        grid=grid,
        in_specs=[row, row, cnt, vec, vec, vec],
        out_specs=pl.BlockSpec((1, 8, Hg), lambda i, j: (i, 0, 0)),
        out_shape=jax.ShapeDtypeStruct((BT, 8, Hg), jnp.float32),
    )(agg, r, cntb, bn, g, b)[:, 0, :]


def _gru_head(hseq, wih_t, whh_t, bih, bhh, head_w, head_b, n_nodes):
    B, T, Hg = hseq.shape
    Ht = whh_t.shape[0]
    full = lambda s: pl.BlockSpec(s, lambda: tuple(0 for _ in s))
    return pl.pallas_call(
        functools.partial(_gru_body, n_nodes=n_nodes),
        in_specs=[full((B, T, Hg)), full((Hg, 3 * Ht)), full((Ht, 3 * Ht)),
                  full((1, 3 * Ht)), full((1, 3 * Ht)), full((Ht, 1)),
                  full((1, 1))],
        out_specs=full((B, 1)),
        out_shape=jax.ShapeDtypeStruct((B, 1), jnp.float32),
    )(hseq, wih_t, whh_t, bih, bhh, head_w, head_b)


_NTILES = 16
_CH = 100
_RPT = 640


def _fill(ref, value):
    rows, cols = ref.shape

    def body(i, _):
        v = jnp.full((16,), value, jnp.float32)
        for j in range(cols // 16):
            ref[i, pl.ds(j * 16, 16)] = v
        return 0

    lax.fori_loop(0, rows, body, 0)


def _tile_slices(s, n):
    row0 = pl.multiple_of(s * _RPT, 8)
    last = n - _RPT * (_NTILES - 1)
    return row0, last


def _zero_slice(zb_v, dst_ref, row0):
    zrows = zb_v.shape[0]
    for j in range(_RPT // zrows):
        pltpu.sync_copy(zb_v, dst_ref.at[pl.ds(row0 + j * zrows, zrows)])


def _write_slice(s, n, acc_ref, hbm_ref):
    row0, last = _tile_slices(s, n)

    @pl.when(s < _NTILES - 1)
    def _():
        pltpu.sync_copy(acc_ref.at[pl.ds(row0, _RPT)],
                        hbm_ref.at[pl.ds(row0, _RPT)])

    @pl.when(s == _NTILES - 1)
    def _():
        base = _RPT * (_NTILES - 1)
        pltpu.sync_copy(acc_ref.at[pl.ds(base, last)],
                        hbm_ref.at[pl.ds(base, last)])


_NBUF = 4


def _agg_body(y_hbm, src3_hbm, dst3_hbm, out_hbm, cnt_hbm, src_v, dst_v,
              rows_v, zb_v, ones_v, acc, *gsems):
    BT, N, W = y_hbm.shape
    nch = src_v.shape[0]
    c = lax.axis_index("c")
    s = lax.axis_index("s")
    row0, _ = _tile_slices(s, N)

    pltpu.sync_copy(src3_hbm.at[s], src_v)
    pltpu.sync_copy(dst3_hbm.at[s], dst_v)
    _fill(zb_v, 0.0)

    @pl.when(c == 0)
    def _():
        _fill(ones_v, 1.0)
        _zero_slice(zb_v, acc, row0)
        plsc.subcore_barrier()

        def cbody(ch, _):
            pltpu.sync_copy(ones_v, acc.at[dst_v.at[ch]], add=True)
            return 0

        lax.fori_loop(0, nch, cbody, 0)
        plsc.subcore_barrier()
        _write_slice(s, N, acc, cnt_hbm)

    n0 = BT // 2 - 1
    start = jnp.where(c == 0, 0, n0)
    npass = jnp.where(c == 0, n0, BT - n0)

    def pass_body(p, _):
        bt = start + p
        _zero_slice(zb_v, acc, row0)
        plsc.subcore_barrier()
        ysrc = y_hbm.at[bt]

        for b in range(_NBUF):
            pltpu.async_copy(ysrc.at[src_v.at[b]], rows_v.at[b], gsems[b])

        def group(g, _):
            base = g * _NBUF
            for b in range(_NBUF):
                ch = base + b
                pltpu.make_async_copy(ysrc.at[src_v.at[ch]], rows_v.at[b],
                                      gsems[b]).wait()
                pltpu.sync_copy(rows_v.at[b], acc.at[dst_v.at[ch]], add=True)
                nxt = ch + _NBUF

                @pl.when(nxt < nch)
                def _():
                    pltpu.async_copy(ysrc.at[src_v.at[nxt]], rows_v.at[b],
                                     gsems[b])
            return 0

        lax.fori_loop(0, nch // _NBUF, group, 0)
        plsc.subcore_barrier()
        _write_slice(s, N, acc, out_hbm.at[bt])
        return 0

    lax.fori_loop(0, npass, pass_body, 0)


@functools.lru_cache(maxsize=None)
def _make_sc_aggregate(bt, n, w, nch):
    mesh = plsc.VectorSubcoreMesh(core_axis_name="c", subcore_axis_name="s")
    n_pad = _RPT * _NTILES
    return functools.partial(
        pl.kernel,
        out_type=[jax.ShapeDtypeStruct((bt, n, w), jnp.float32),
                  jax.ShapeDtypeStruct((n, w), jnp.float32)],
        scratch_types=[
            pltpu.VMEM((nch, _CH), jnp.int32),
            pltpu.VMEM((nch, _CH), jnp.int32),
            pltpu.VMEM((_NBUF, _CH, w), jnp.float32),
            pltpu.VMEM((80, w), jnp.float32),
            pltpu.VMEM((_CH, w), jnp.float32),
            pltpu.VMEM_SHARED((n_pad, w), jnp.float32),
        ] + [pltpu.SemaphoreType.DMA] * _NBUF,
        mesh=mesh,
        compiler_params=pltpu.CompilerParams(use_tc_tiling_on_sc=False),
    )(_agg_body)


def _sc_aggregate(y, src3, dst3):
    BT, N, W = y.shape
    return _make_sc_aggregate(BT, N, W, src3.shape[1])(y, src3, dst3)


def kernel(x_seq, edge_index, W_neigh0, b_neigh0, W_root0, ln_g0, ln_b0,
           W_neigh1, b_neigh1, W_root1, ln_g1, ln_b1,
           gru_Wih, gru_Whh, gru_bih, gru_bhh, head_W, head_b):
    B, T, N, F = x_seq.shape
    E = edge_index.shape[1]
    Hg = W_neigh0.shape[1]
    Ht = gru_Whh.shape[1]
    BT = B * T
    M = BT * N

    ept = E // _NTILES
    src3 = edge_index[0].astype(jnp.int32).reshape(_NTILES, ept // _CH, _CH)
    dst3 = edge_index[1].astype(jnp.int32).reshape(_NTILES, ept // _CH, _CH)

    x = x_seq.reshape(M, F)
    P0, R0 = _project(x, W_neigh0, W_root0, mb=2000)

    agg0, cntb = _sc_aggregate(P0.reshape(BT, N, Hg), src3, dst3)

    P1, R1 = _fused_ln_proj(agg0, R0.reshape(BT, N, Hg), cntb,
                            b_neigh0.reshape(1, Hg), ln_g0.reshape(1, Hg),
                            ln_b0.reshape(1, Hg), W_neigh1, W_root1, nb=1000)

    agg1, _cnt_unused = _sc_aggregate(P1, src3, dst3)

    hsum = _ln_mean(agg1, R1, cntb, b_neigh1.reshape(1, Hg),
                    ln_g1.reshape(1, Hg), ln_b1.reshape(1, Hg), nb=1000)

    y = _gru_head(hsum.reshape(B, T, Hg), gru_Wih.T, gru_Whh.T,
                  gru_bih.reshape(1, 3 * Ht), gru_bhh.reshape(1, 3 * Ht),
                  head_W, head_b.reshape(1, 1), n_nodes=N)
    return y[:, 0]


# speedup vs baseline: 20.9382x; 1.0029x over previous
import functools

import jax
import jax.numpy as jnp
from jax import lax
from jax.experimental import pallas as pl
from jax.experimental.pallas import tpu as pltpu
from jax.experimental.pallas import tpu_sc as plsc


def _proj_body(x_ref, wn_ref, wr_ref, p_ref, r_ref):
    x = x_ref[...]
    p_ref[...] = jnp.dot(x, wn_ref[...], preferred_element_type=jnp.float32)
    r_ref[...] = jnp.dot(x, wr_ref[...], preferred_element_type=jnp.float32)


def _ln_from_parts(agg, r, cnt, bn, g, b):
    inv = 1.0 / jnp.maximum(cnt[:, 0:1], 1.0)
    z = agg * inv + bn + r
    mu = jnp.mean(z, axis=-1, keepdims=True)
    var = jnp.mean((z - mu) ** 2, axis=-1, keepdims=True)
    h = (z - mu) * lax.rsqrt(var + 1e-5) * g + b
    return jnp.maximum(h, 0.0)


def _fused_ln_proj_body(agg_ref, r_ref, cnt_ref, bn_ref, g_ref, b_ref,
                        wn_ref, wr_ref, p1_ref, r1_ref):
    h = _ln_from_parts(agg_ref[0], r_ref[0], cnt_ref[...],
                       bn_ref[...], g_ref[...], b_ref[...])
    p1_ref[0] = jnp.dot(h, wn_ref[...], preferred_element_type=jnp.float32)
    r1_ref[0] = jnp.dot(h, wr_ref[...], preferred_element_type=jnp.float32)


def _ln_mean_body(agg_ref, r_ref, cnt_ref, bn_ref, g_ref, b_ref, hsum_ref):
    nb = pl.program_id(1)
    h = _ln_from_parts(agg_ref[0], r_ref[0], cnt_ref[...],
                       bn_ref[...], g_ref[...], b_ref[...])

    @pl.when(nb == 0)
    def _():
        hsum_ref[...] = jnp.zeros_like(hsum_ref)

    s = jnp.sum(h, axis=0)
    hsum_ref[...] += jnp.broadcast_to(s[None, None, :], hsum_ref.shape)


def _gru_body(hseq_ref, wih_ref, whh_ref, bih_ref, bhh_ref, hw_ref, hb_ref,
              y_ref, *, n_nodes):
    B, T, _ = hseq_ref.shape
    Ht = whh_ref.shape[0]
    h = jnp.zeros((B, Ht), jnp.float32)
    scale = 1.0 / float(n_nodes)
    for t in range(T):
        ht = hseq_ref[:, t, :] * scale
        gi = jnp.dot(ht, wih_ref[...], preferred_element_type=jnp.float32) + bih_ref[...]
        gh = jnp.dot(h, whh_ref[...], preferred_element_type=jnp.float32) + bhh_ref[...]
        ir, iz, inn = gi[:, :Ht], gi[:, Ht:2 * Ht], gi[:, 2 * Ht:]
        hr, hz, hn = gh[:, :Ht], gh[:, Ht:2 * Ht], gh[:, 2 * Ht:]
        r = jax.nn.sigmoid(ir + hr)
        z = jax.nn.sigmoid(iz + hz)
        n = jnp.tanh(inn + r * hn)
        h = (1.0 - z) * n + z * h
    y_ref[...] = jnp.dot(h, hw_ref[...], preferred_element_type=jnp.float32) + hb_ref[...]


def _project(x, wn, wr, mb):
    M, F = x.shape
    Hg = wn.shape[1]
    return pl.pallas_call(
        _proj_body,
        grid=(M // mb,),
        in_specs=[
            pl.BlockSpec((mb, F), lambda i: (i, 0)),
            pl.BlockSpec((F, Hg), lambda i: (0, 0)),
            pl.BlockSpec((F, Hg), lambda i: (0, 0)),
        ],
        out_specs=[
            pl.BlockSpec((mb, Hg), lambda i: (i, 0)),
            pl.BlockSpec((mb, Hg), lambda i: (i, 0)),
        ],
        out_shape=[
            jax.ShapeDtypeStruct((M, Hg), jnp.float32),
            jax.ShapeDtypeStruct((M, Hg), jnp.float32),
        ],
    )(x, wn, wr)


def _fused_ln_proj(agg, r, cntb, bn, g, b, wn, wr, nb):
    BT, N, Hg = agg.shape
    grid = (BT, N // nb)
    row = pl.BlockSpec((1, nb, Hg), lambda i, j: (i, j, 0))
    vec = pl.BlockSpec((1, Hg), lambda i, j: (0, 0))
    mat = pl.BlockSpec((Hg, Hg), lambda i, j: (0, 0))
    cnt = pl.BlockSpec((nb, cntb.shape[1]), lambda i, j: (j, 0))
    return pl.pallas_call(
        _fused_ln_proj_body,
        grid=grid,
        in_specs=[row, row, cnt, vec, vec, vec, mat, mat],
        out_specs=[row, row],
        out_shape=[
            jax.ShapeDtypeStruct((BT, N, Hg), jnp.float32),
            jax.ShapeDtypeStruct((BT, N, Hg), jnp.float32),
        ],
    )(agg, r, cntb, bn, g, b, wn, wr)


def _ln_mean(agg, r, cntb, bn, g, b, nb):
    BT, N, Hg = agg.shape
    grid = (BT, N // nb)
    row = pl.BlockSpec((1, nb, Hg), lambda i, j: (i, j, 0))
    vec = pl.BlockSpec((1, Hg), lambda i, j: (0, 0))
    cnt = pl.BlockSpec((nb, cntb.shape[1]), lambda i, j: (j, 0))
    return pl.pallas_call(
        _ln_mean_body,
        grid=grid,
        in_specs=[row, row, cnt, vec, vec, vec],
        out_specs=pl.BlockSpec((1, 8, Hg), lambda i, j: (i, 0, 0)),
        out_shape=jax.ShapeDtypeStruct((BT, 8, Hg), jnp.float32),
    )(agg, r, cntb, bn, g, b)[:, 0, :]


def _gru_head(hseq, wih_t, whh_t, bih, bhh, head_w, head_b, n_nodes):
    B, T, Hg = hseq.shape
    Ht = whh_t.shape[0]
    full = lambda s: pl.BlockSpec(s, lambda: tuple(0 for _ in s))
    return pl.pallas_call(
        functools.partial(_gru_body, n_nodes=n_nodes),
        in_specs=[full((B, T, Hg)), full((Hg, 3 * Ht)), full((Ht, 3 * Ht)),
                  full((1, 3 * Ht)), full((1, 3 * Ht)), full((Ht, 1)),
                  full((1, 1))],
        out_specs=full((B, 1)),
        out_shape=jax.ShapeDtypeStruct((B, 1), jnp.float32),
    )(hseq, wih_t, whh_t, bih, bhh, head_w, head_b)


_NTILES = 16
_CH = 100
_RPT = 640


def _fill(ref, value):
    rows, cols = ref.shape

    def body(i, _):
        v = jnp.full((16,), value, jnp.float32)
        for j in range(cols // 16):
            ref[i, pl.ds(j * 16, 16)] = v
        return 0

    lax.fori_loop(0, rows, body, 0)


def _tile_slices(s, n):
    row0 = pl.multiple_of(s * _RPT, 8)
    last = n - _RPT * (_NTILES - 1)
    return row0, last


def _zero_slice(zb_v, dst_ref, row0):
    zrows = zb_v.shape[0]
    for j in range(_RPT // zrows):
        pltpu.sync_copy(zb_v, dst_ref.at[pl.ds(row0 + j * zrows, zrows)])


def _write_slice(s, n, acc_ref, hbm_ref):
    row0, last = _tile_slices(s, n)

    @pl.when(s < _NTILES - 1)
    def _():
        pltpu.sync_copy(acc_ref.at[pl.ds(row0, _RPT)],
                        hbm_ref.at[pl.ds(row0, _RPT)])

    @pl.when(s == _NTILES - 1)
    def _():
        base = _RPT * (_NTILES - 1)
        pltpu.sync_copy(acc_ref.at[pl.ds(base, last)],
                        hbm_ref.at[pl.ds(base, last)])


_NBUF = 5


def _agg_body(y_hbm, src3_hbm, dst3_hbm, out_hbm, cnt_hbm, src_v, dst_v,
              rows_v, zb_v, ones_v, acc, *gsems):
    BT, N, W = y_hbm.shape
    nch = src_v.shape[0]
    c = lax.axis_index("c")
    s = lax.axis_index("s")
    row0, _ = _tile_slices(s, N)

    pltpu.sync_copy(src3_hbm.at[s], src_v)
    pltpu.sync_copy(dst3_hbm.at[s], dst_v)
    _fill(zb_v, 0.0)

    @pl.when(c == 0)
    def _():
        _fill(ones_v, 1.0)
        _zero_slice(zb_v, acc, row0)
        plsc.subcore_barrier()

        def cbody(ch, _):
            pltpu.sync_copy(ones_v, acc.at[dst_v.at[ch]], add=True)
            return 0

        lax.fori_loop(0, nch, cbody, 0)
        plsc.subcore_barrier()
        _write_slice(s, N, acc, cnt_hbm)

    n0 = BT // 2 - 1
    start = jnp.where(c == 0, 0, n0)
    npass = jnp.where(c == 0, n0, BT - n0)

    def pass_body(p, _):
        bt = start + p
        _zero_slice(zb_v, acc, row0)
        plsc.subcore_barrier()
        ysrc = y_hbm.at[bt]

        for b in range(_NBUF):
            pltpu.async_copy(ysrc.at[src_v.at[b]], rows_v.at[b], gsems[b])

        def group(g, _):
            base = g * _NBUF
            for b in range(_NBUF):
                ch = base + b
                pltpu.make_async_copy(ysrc.at[src_v.at[ch]], rows_v.at[b],
                                      gsems[b]).wait()
                pltpu.sync_copy(rows_v.at[b], acc.at[dst_v.at[ch]], add=True)
                nxt = ch + _NBUF

                @pl.when(nxt < nch)
                def _():
                    pltpu.async_copy(ysrc.at[src_v.at[nxt]], rows_v.at[b],
                                     gsems[b])
            return 0

        lax.fori_loop(0, nch // _NBUF, group, 0)
        plsc.subcore_barrier()
        _write_slice(s, N, acc, out_hbm.at[bt])
        return 0

    lax.fori_loop(0, npass, pass_body, 0)


@functools.lru_cache(maxsize=None)
def _make_sc_aggregate(bt, n, w, nch):
    mesh = plsc.VectorSubcoreMesh(core_axis_name="c", subcore_axis_name="s")
    n_pad = _RPT * _NTILES
    return functools.partial(
        pl.kernel,
        out_type=[jax.ShapeDtypeStruct((bt, n, w), jnp.float32),
                  jax.ShapeDtypeStruct((n, w), jnp.float32)],
        scratch_types=[
            pltpu.VMEM((nch, _CH), jnp.int32),
            pltpu.VMEM((nch, _CH), jnp.int32),
            pltpu.VMEM((_NBUF, _CH, w), jnp.float32),
            pltpu.VMEM((80, w), jnp.float32),
            pltpu.VMEM((_CH, w), jnp.float32),
            pltpu.VMEM_SHARED((n_pad, w), jnp.float32),
        ] + [pltpu.SemaphoreType.DMA] * _NBUF,
        mesh=mesh,
        compiler_params=pltpu.CompilerParams(use_tc_tiling_on_sc=False),
    )(_agg_body)


def _sc_aggregate(y, src3, dst3):
    BT, N, W = y.shape
    return _make_sc_aggregate(BT, N, W, src3.shape[1])(y, src3, dst3)


def kernel(x_seq, edge_index, W_neigh0, b_neigh0, W_root0, ln_g0, ln_b0,
           W_neigh1, b_neigh1, W_root1, ln_g1, ln_b1,
           gru_Wih, gru_Whh, gru_bih, gru_bhh, head_W, head_b):
    B, T, N, F = x_seq.shape
    E = edge_index.shape[1]
    Hg = W_neigh0.shape[1]
    Ht = gru_Whh.shape[1]
    BT = B * T
    M = BT * N

    ept = E // _NTILES
    src3 = edge_index[0].astype(jnp.int32).reshape(_NTILES, ept // _CH, _CH)
    dst3 = edge_index[1].astype(jnp.int32).reshape(_NTILES, ept // _CH, _CH)

    x = x_seq.reshape(M, F)
    P0, R0 = _project(x, W_neigh0, W_root0, mb=2000)

    agg0, cntb = _sc_aggregate(P0.reshape(BT, N, Hg), src3, dst3)

    P1, R1 = _fused_ln_proj(agg0, R0.reshape(BT, N, Hg), cntb,
                            b_neigh0.reshape(1, Hg), ln_g0.reshape(1, Hg),
                            ln_b0.reshape(1, Hg), W_neigh1, W_root1, nb=1000)

    agg1, _cnt_unused = _sc_aggregate(P1, src3, dst3)

    hsum = _ln_mean(agg1, R1, cntb, b_neigh1.reshape(1, Hg),
                    ln_g1.reshape(1, Hg), ln_b1.reshape(1, Hg), nb=1000)

    y = _gru_head(hsum.reshape(B, T, Hg), gru_Wih.T, gru_Whh.T,
                  gru_bih.reshape(1, 3 * Ht), gru_bhh.reshape(1, 3 * Ht),
                  head_W, head_b.reshape(1, 1), n_nodes=N)
    return y[:, 0]


# speedup vs baseline: 22.3577x; 1.0678x over previous
import functools

import jax
import jax.numpy as jnp
from jax import lax
from jax.experimental import pallas as pl
from jax.experimental.pallas import tpu as pltpu
from jax.experimental.pallas import tpu_sc as plsc


def _proj_body(x_ref, wn_ref, wr_ref, p_ref, r_ref):
    x = x_ref[...]
    p_ref[...] = jnp.dot(x, wn_ref[...], preferred_element_type=jnp.float32)
    r_ref[...] = jnp.dot(x, wr_ref[...], preferred_element_type=jnp.float32)


def _ln_from_parts(agg, r, cnt, bn, g, b):
    inv = 1.0 / jnp.maximum(cnt[:, 0:1], 1.0)
    z = agg * inv + bn + r
    mu = jnp.mean(z, axis=-1, keepdims=True)
    var = jnp.mean((z - mu) ** 2, axis=-1, keepdims=True)
    h = (z - mu) * lax.rsqrt(var + 1e-5) * g + b
    return jnp.maximum(h, 0.0)


def _fused_ln_proj_body(agg_ref, r_ref, cnt_ref, bn_ref, g_ref, b_ref,
                        wn_ref, wr_ref, p1_ref, r1_ref):
    h = _ln_from_parts(agg_ref[0], r_ref[0], cnt_ref[...],
                       bn_ref[...], g_ref[...], b_ref[...])
    p1_ref[0] = jnp.dot(h, wn_ref[...], preferred_element_type=jnp.float32)
    r1_ref[0] = jnp.dot(h, wr_ref[...], preferred_element_type=jnp.float32)


def _ln_mean_body(agg_ref, r_ref, cnt_ref, bn_ref, g_ref, b_ref, hsum_ref):
    nb = pl.program_id(1)
    h = _ln_from_parts(agg_ref[0], r_ref[0], cnt_ref[...],
                       bn_ref[...], g_ref[...], b_ref[...])

    @pl.when(nb == 0)
    def _():
        hsum_ref[...] = jnp.zeros_like(hsum_ref)

    s = jnp.sum(h, axis=0)
    hsum_ref[...] += jnp.broadcast_to(s[None, None, :], hsum_ref.shape)


def _gru_body(hseq_ref, wih_ref, whh_ref, bih_ref, bhh_ref, hw_ref, hb_ref,
              y_ref, *, n_nodes):
    B, T, _ = hseq_ref.shape
    Ht = whh_ref.shape[0]
    h = jnp.zeros((B, Ht), jnp.float32)
    scale = 1.0 / float(n_nodes)
    for t in range(T):
        ht = hseq_ref[:, t, :] * scale
        gi = jnp.dot(ht, wih_ref[...], preferred_element_type=jnp.float32) + bih_ref[...]
        gh = jnp.dot(h, whh_ref[...], preferred_element_type=jnp.float32) + bhh_ref[...]
        ir, iz, inn = gi[:, :Ht], gi[:, Ht:2 * Ht], gi[:, 2 * Ht:]
        hr, hz, hn = gh[:, :Ht], gh[:, Ht:2 * Ht], gh[:, 2 * Ht:]
        r = jax.nn.sigmoid(ir + hr)
        z = jax.nn.sigmoid(iz + hz)
        n = jnp.tanh(inn + r * hn)
        h = (1.0 - z) * n + z * h
    y_ref[...] = jnp.dot(h, hw_ref[...], preferred_element_type=jnp.float32) + hb_ref[...]


def _project(x, wn, wr, mb):
    M, F = x.shape
    Hg = wn.shape[1]
    return pl.pallas_call(
        _proj_body,
        grid=(M // mb,),
        in_specs=[
            pl.BlockSpec((mb, F), lambda i: (i, 0)),
            pl.BlockSpec((F, Hg), lambda i: (0, 0)),
            pl.BlockSpec((F, Hg), lambda i: (0, 0)),
        ],
        out_specs=[
            pl.BlockSpec((mb, Hg), lambda i: (i, 0)),
            pl.BlockSpec((mb, Hg), lambda i: (i, 0)),
        ],
        out_shape=[
            jax.ShapeDtypeStruct((M, Hg), jnp.float32),
            jax.ShapeDtypeStruct((M, Hg), jnp.float32),
        ],
    )(x, wn, wr)


def _fused_ln_proj(agg, r, cntb, bn, g, b, wn, wr, nb):
    BT, N, Hg = agg.shape
    grid = (BT, N // nb)
    row = pl.BlockSpec((1, nb, Hg), lambda i, j: (i, j, 0))
    vec = pl.BlockSpec((1, Hg), lambda i, j: (0, 0))
    mat = pl.BlockSpec((Hg, Hg), lambda i, j: (0, 0))
    cnt = pl.BlockSpec((nb, cntb.shape[1]), lambda i, j: (j, 0))
    return pl.pallas_call(
        _fused_ln_proj_body,
        grid=grid,
        in_specs=[row, row, cnt, vec, vec, vec, mat, mat],
        out_specs=[row, row],
        out_shape=[
            jax.ShapeDtypeStruct((BT, N, Hg), jnp.float32),
            jax.ShapeDtypeStruct((BT, N, Hg), jnp.float32),
        ],
    )(agg, r, cntb, bn, g, b, wn, wr)


def _ln_mean(agg, r, cntb, bn, g, b, nb):
    BT, N, Hg = agg.shape
    grid = (BT, N // nb)
    row = pl.BlockSpec((1, nb, Hg), lambda i, j: (i, j, 0))
    vec = pl.BlockSpec((1, Hg), lambda i, j: (0, 0))
    cnt = pl.BlockSpec((nb, cntb.shape[1]), lambda i, j: (j, 0))
    return pl.pallas_call(
        _ln_mean_body,
        grid=grid,
        in_specs=[row, row, cnt, vec, vec, vec],
        out_specs=pl.BlockSpec((1, 8, Hg), lambda i, j: (i, 0, 0)),
        out_shape=jax.ShapeDtypeStruct((BT, 8, Hg), jnp.float32),
    )(agg, r, cntb, bn, g, b)[:, 0, :]


def _gru_head(hseq, wih_t, whh_t, bih, bhh, head_w, head_b, n_nodes):
    B, T, Hg = hseq.shape
    Ht = whh_t.shape[0]
    full = lambda s: pl.BlockSpec(s, lambda: tuple(0 for _ in s))
    return pl.pallas_call(
        functools.partial(_gru_body, n_nodes=n_nodes),
        in_specs=[full((B, T, Hg)), full((Hg, 3 * Ht)), full((Ht, 3 * Ht)),
                  full((1, 3 * Ht)), full((1, 3 * Ht)), full((Ht, 1)),
                  full((1, 1))],
        out_specs=full((B, 1)),
        out_shape=jax.ShapeDtypeStruct((B, 1), jnp.float32),
    )(hseq, wih_t, whh_t, bih, bhh, head_w, head_b)


_NTILES = 16
_CH = 100
_RPT = 640


def _fill(ref, value):
    rows, cols = ref.shape

    def body(i, _):
        v = jnp.full((16,), value, jnp.float32)
        for j in range(cols // 16):
            ref[i, pl.ds(j * 16, 16)] = v
        return 0

    lax.fori_loop(0, rows, body, 0)


def _tile_slices(s, n):
    row0 = pl.multiple_of(s * _RPT, 8)
    last = n - _RPT * (_NTILES - 1)
    return row0, last


def _zero_slice(zb_v, dst_ref, row0):
    zrows = zb_v.shape[0]
    for j in range(_RPT // zrows):
        pltpu.sync_copy(zb_v, dst_ref.at[pl.ds(row0 + j * zrows, zrows)])


def _write_slice(s, n, acc_ref, hbm_ref):
    row0, last = _tile_slices(s, n)

    @pl.when(s < _NTILES - 1)
    def _():
        pltpu.sync_copy(acc_ref.at[pl.ds(row0, _RPT)],
                        hbm_ref.at[pl.ds(row0, _RPT)])

    @pl.when(s == _NTILES - 1)
    def _():
        base = _RPT * (_NTILES - 1)
        pltpu.sync_copy(acc_ref.at[pl.ds(base, last)],
                        hbm_ref.at[pl.ds(base, last)])


_NBUF = 5


def _agg_body(y_hbm, src3_hbm, dst3_hbm, out_hbm, cnt_hbm, src_v, dst_v,
              rows_v, zb_v, ones_v, acc, *gsems):
    BT, N, W = y_hbm.shape
    nch = src_v.shape[0]
    c = lax.axis_index("c")
    s = lax.axis_index("s")
    row0, _ = _tile_slices(s, N)

    pltpu.sync_copy(src3_hbm.at[s], src_v)
    pltpu.sync_copy(dst3_hbm.at[s], dst_v)
    _fill(zb_v, 0.0)

    @pl.when(c == 0)
    def _():
        _fill(ones_v, 1.0)
        _zero_slice(zb_v, acc, row0)
        plsc.subcore_barrier()

        def cbody(ch, _):
            pltpu.sync_copy(ones_v, acc.at[dst_v.at[ch]], add=True)
            return 0

        lax.fori_loop(0, nch, cbody, 0)
        plsc.subcore_barrier()
        _write_slice(s, N, acc, cnt_hbm)

    n0 = BT // 2 - 1
    start = jnp.where(c == 0, 0, n0)
    npass = jnp.where(c == 0, n0, BT - n0)

    def pass_body(p, _):
        bt = start + p
        _zero_slice(zb_v, acc, row0)
        plsc.subcore_barrier()
        ysrc = y_hbm.at[bt]

        for b in range(_NBUF):
            pltpu.async_copy(ysrc.at[src_v.at[b]], rows_v.at[b], gsems[b])

        def group(g, _):
            base = g * _NBUF
            for b in range(_NBUF):
                ch = base + b
                pltpu.make_async_copy(ysrc.at[src_v.at[ch]], rows_v.at[b],
                                      gsems[b]).wait()
                pltpu.sync_copy(rows_v.at[b], acc.at[dst_v.at[ch]], add=True)
                nxt = ch + _NBUF

                @pl.when(nxt < nch)
                def _():
                    pltpu.async_copy(ysrc.at[src_v.at[nxt]], rows_v.at[b],
                                     gsems[b])
            return 0

        lax.fori_loop(0, nch // _NBUF, group, 0)
        plsc.subcore_barrier()
        _write_slice(s, N, acc, out_hbm.at[bt])
        return 0

    lax.fori_loop(0, npass, pass_body, 0)


@functools.lru_cache(maxsize=None)
def _make_sc_aggregate(bt, n, w, nch):
    mesh = plsc.VectorSubcoreMesh(core_axis_name="c", subcore_axis_name="s")
    n_pad = _RPT * _NTILES
    return functools.partial(
        pl.kernel,
        out_type=[jax.ShapeDtypeStruct((bt, n, w), jnp.float32),
                  jax.ShapeDtypeStruct((n, w), jnp.float32)],
        scratch_types=[
            pltpu.VMEM((nch, _CH), jnp.int32),
            pltpu.VMEM((nch, _CH), jnp.int32),
            pltpu.VMEM((_NBUF, _CH, w), jnp.float32),
            pltpu.VMEM((80, w), jnp.float32),
            pltpu.VMEM((_CH, w), jnp.float32),
            pltpu.VMEM_SHARED((n_pad, w), jnp.float32),
        ] + [pltpu.SemaphoreType.DMA] * _NBUF,
        mesh=mesh,
        compiler_params=pltpu.CompilerParams(use_tc_tiling_on_sc=False),
    )(_agg_body)


def _sc_aggregate(y, src3, dst3):
    BT, N, W = y.shape
    return _make_sc_aggregate(BT, N, W, src3.shape[1])(y, src3, dst3)


def kernel(x_seq, edge_index, W_neigh0, b_neigh0, W_root0, ln_g0, ln_b0,
           W_neigh1, b_neigh1, W_root1, ln_g1, ln_b1,
           gru_Wih, gru_Whh, gru_bih, gru_bhh, head_W, head_b):
    B, T, N, F = x_seq.shape
    E = edge_index.shape[1]
    Hg = W_neigh0.shape[1]
    Ht = gru_Whh.shape[1]
    BT = B * T
    M = BT * N

    ept = E // _NTILES
    src3 = edge_index[0].astype(jnp.int32).reshape(_NTILES, ept // _CH, _CH)
    dst3 = edge_index[1].astype(jnp.int32).reshape(_NTILES, ept // _CH, _CH)

    x = x_seq.reshape(M, F)
    P0, R0 = _project(x, W_neigh0, W_root0, mb=2000)

    agg0, cntb = _sc_aggregate(P0.reshape(BT, N, Hg), src3, dst3)

    P1, R1 = _fused_ln_proj(agg0, R0.reshape(BT, N, Hg), cntb,
                            b_neigh0.reshape(1, Hg), ln_g0.reshape(1, Hg),
                            ln_b0.reshape(1, Hg), W_neigh1, W_root1, nb=2000)

    agg1, _cnt_unused = _sc_aggregate(P1, src3, dst3)

    hsum = _ln_mean(agg1, R1, cntb, b_neigh1.reshape(1, Hg),
                    ln_g1.reshape(1, Hg), ln_b1.reshape(1, Hg), nb=2000)

    y = _gru_head(hsum.reshape(B, T, Hg), gru_Wih.T, gru_Whh.T,
                  gru_bih.reshape(1, 3 * Ht), gru_bhh.reshape(1, 3 * Ht),
                  head_W, head_b.reshape(1, 1), n_nodes=N)
    return y[:, 0]


# speedup vs baseline: 23.8503x; 1.0668x over previous
import functools

import jax
import jax.numpy as jnp
from jax import lax
from jax.experimental import pallas as pl
from jax.experimental.pallas import tpu as pltpu
from jax.experimental.pallas import tpu_sc as plsc


def _proj_body(x_ref, wn_ref, wr_ref, p_ref, r_ref):
    x = x_ref[...]
    p_ref[...] = jnp.dot(x, wn_ref[...], preferred_element_type=jnp.float32)
    r_ref[...] = jnp.dot(x, wr_ref[...], preferred_element_type=jnp.float32)


def _ln_from_parts(agg, r, cnt, bn, g, b):
    inv = 1.0 / jnp.maximum(cnt[:, 0:1], 1.0)
    z = agg * inv + bn + r
    mu = jnp.mean(z, axis=-1, keepdims=True)
    var = jnp.mean((z - mu) ** 2, axis=-1, keepdims=True)
    h = (z - mu) * lax.rsqrt(var + 1e-5) * g + b
    return jnp.maximum(h, 0.0)


def _fused_ln_proj_body(agg_ref, r_ref, cnt_ref, bn_ref, g_ref, b_ref,
                        wn_ref, wr_ref, p1_ref, r1_ref):
    h = _ln_from_parts(agg_ref[0], r_ref[0], cnt_ref[...],
                       bn_ref[...], g_ref[...], b_ref[...])
    p1_ref[0] = jnp.dot(h, wn_ref[...], preferred_element_type=jnp.float32)
    r1_ref[0] = jnp.dot(h, wr_ref[...], preferred_element_type=jnp.float32)


def _ln_mean_body(agg_ref, r_ref, cnt_ref, bn_ref, g_ref, b_ref, hsum_ref):
    nb = pl.program_id(1)
    h = _ln_from_parts(agg_ref[0], r_ref[0], cnt_ref[...],
                       bn_ref[...], g_ref[...], b_ref[...])

    @pl.when(nb == 0)
    def _():
        hsum_ref[...] = jnp.zeros_like(hsum_ref)

    s = jnp.sum(h, axis=0)
    hsum_ref[...] += jnp.broadcast_to(s[None, None, :], hsum_ref.shape)


def _gru_body(hseq_ref, wih_ref, whh_ref, bih_ref, bhh_ref, hw_ref, hb_ref,
              y_ref, *, n_nodes):
    B, T, _ = hseq_ref.shape
    Ht = whh_ref.shape[0]
    h = jnp.zeros((B, Ht), jnp.float32)
    scale = 1.0 / float(n_nodes)
    for t in range(T):
        ht = hseq_ref[:, t, :] * scale
        gi = jnp.dot(ht, wih_ref[...], preferred_element_type=jnp.float32) + bih_ref[...]
        gh = jnp.dot(h, whh_ref[...], preferred_element_type=jnp.float32) + bhh_ref[...]
        ir, iz, inn = gi[:, :Ht], gi[:, Ht:2 * Ht], gi[:, 2 * Ht:]
        hr, hz, hn = gh[:, :Ht], gh[:, Ht:2 * Ht], gh[:, 2 * Ht:]
        r = jax.nn.sigmoid(ir + hr)
        z = jax.nn.sigmoid(iz + hz)
        n = jnp.tanh(inn + r * hn)
        h = (1.0 - z) * n + z * h
    y_ref[...] = jnp.dot(h, hw_ref[...], preferred_element_type=jnp.float32) + hb_ref[...]


def _project(x, wn, wr, mb):
    M, F = x.shape
    Hg = wn.shape[1]
    return pl.pallas_call(
        _proj_body,
        grid=(M // mb,),
        in_specs=[
            pl.BlockSpec((mb, F), lambda i: (i, 0)),
            pl.BlockSpec((F, Hg), lambda i: (0, 0)),
            pl.BlockSpec((F, Hg), lambda i: (0, 0)),
        ],
        out_specs=[
            pl.BlockSpec((mb, Hg), lambda i: (i, 0)),
            pl.BlockSpec((mb, Hg), lambda i: (i, 0)),
        ],
        out_shape=[
            jax.ShapeDtypeStruct((M, Hg), jnp.float32),
            jax.ShapeDtypeStruct((M, Hg), jnp.float32),
        ],
    )(x, wn, wr)


def _fused_ln_proj(agg, r, cntb, bn, g, b, wn, wr, nb):
    BT, N, Hg = agg.shape
    grid = (BT, N // nb)
    row = pl.BlockSpec((1, nb, Hg), lambda i, j: (i, j, 0))
    vec = pl.BlockSpec((1, Hg), lambda i, j: (0, 0))
    mat = pl.BlockSpec((Hg, Hg), lambda i, j: (0, 0))
    cnt = pl.BlockSpec((nb, cntb.shape[1]), lambda i, j: (j, 0))
    return pl.pallas_call(
        _fused_ln_proj_body,
        grid=grid,
        in_specs=[row, row, cnt, vec, vec, vec, mat, mat],
        out_specs=[row, row],
        out_shape=[
            jax.ShapeDtypeStruct((BT, N, Hg), jnp.float32),
            jax.ShapeDtypeStruct((BT, N, Hg), jnp.float32),
        ],
    )(agg, r, cntb, bn, g, b, wn, wr)


def _ln_mean(agg, r, cntb, bn, g, b, nb):
    BT, N, Hg = agg.shape
    grid = (BT, N // nb)
    row = pl.BlockSpec((1, nb, Hg), lambda i, j: (i, j, 0))
    vec = pl.BlockSpec((1, Hg), lambda i, j: (0, 0))
    cnt = pl.BlockSpec((nb, cntb.shape[1]), lambda i, j: (j, 0))
    return pl.pallas_call(
        _ln_mean_body,
        grid=grid,
        in_specs=[row, row, cnt, vec, vec, vec],
        out_specs=pl.BlockSpec((1, 8, Hg), lambda i, j: (i, 0, 0)),
        out_shape=jax.ShapeDtypeStruct((BT, 8, Hg), jnp.float32),
    )(agg, r, cntb, bn, g, b)[:, 0, :]


def _gru_head(hseq, wih_t, whh_t, bih, bhh, head_w, head_b, n_nodes):
    B, T, Hg = hseq.shape
    Ht = whh_t.shape[0]
    full = lambda s: pl.BlockSpec(s, lambda: tuple(0 for _ in s))
    return pl.pallas_call(
        functools.partial(_gru_body, n_nodes=n_nodes),
        in_specs=[full((B, T, Hg)), full((Hg, 3 * Ht)), full((Ht, 3 * Ht)),
                  full((1, 3 * Ht)), full((1, 3 * Ht)), full((Ht, 1)),
                  full((1, 1))],
        out_specs=full((B, 1)),
        out_shape=jax.ShapeDtypeStruct((B, 1), jnp.float32),
    )(hseq, wih_t, whh_t, bih, bhh, head_w, head_b)


_NTILES = 16
_CH = 100
_RPT = 640


def _fill(ref, value):
    rows, cols = ref.shape

    def body(i, _):
        v = jnp.full((16,), value, jnp.float32)
        for j in range(cols // 16):
            ref[i, pl.ds(j * 16, 16)] = v
        return 0

    lax.fori_loop(0, rows, body, 0)


def _tile_slices(s, n):
    row0 = pl.multiple_of(s * _RPT, 8)
    last = n - _RPT * (_NTILES - 1)
    return row0, last


def _zero_slice(zb_v, dst_ref, row0):
    zrows = zb_v.shape[0]
    for j in range(_RPT // zrows):
        pltpu.sync_copy(zb_v, dst_ref.at[pl.ds(row0 + j * zrows, zrows)])


def _write_slice(s, n, acc_ref, hbm_ref):
    row0, last = _tile_slices(s, n)

    @pl.when(s < _NTILES - 1)
    def _():
        pltpu.sync_copy(acc_ref.at[pl.ds(row0, _RPT)],
                        hbm_ref.at[pl.ds(row0, _RPT)])

    @pl.when(s == _NTILES - 1)
    def _():
        base = _RPT * (_NTILES - 1)
        pltpu.sync_copy(acc_ref.at[pl.ds(base, last)],
                        hbm_ref.at[pl.ds(base, last)])


_NBUF = 5


def _agg_body(make_cnt, y_hbm, src3_hbm, dst3_hbm, out_hbm, *rest):
    if make_cnt:
        cnt_hbm, src_v, dst_v, rows_v, zb_v, ones_v, acc = rest[:7]
        gsems = rest[7:]
    else:
        src_v, dst_v, rows_v, zb_v, acc = rest[:5]
        gsems = rest[5:]
    BT, N, W = y_hbm.shape
    nch = src_v.shape[0]
    c = lax.axis_index("c")
    s = lax.axis_index("s")
    row0, _ = _tile_slices(s, N)

    pltpu.sync_copy(src3_hbm.at[s], src_v)
    pltpu.sync_copy(dst3_hbm.at[s], dst_v)
    _fill(zb_v, 0.0)

    if make_cnt:
        @pl.when(c == 0)
        def _():
            _fill(ones_v, 1.0)
            _zero_slice(zb_v, acc, row0)
            plsc.subcore_barrier()

            def cbody(ch, _):
                pltpu.sync_copy(ones_v, acc.at[dst_v.at[ch]], add=True)
                return 0

            lax.fori_loop(0, nch, cbody, 0)
            plsc.subcore_barrier()
            _write_slice(s, N, acc, cnt_hbm)

    n0 = BT // 2 - (1 if make_cnt else 0)
    start = jnp.where(c == 0, 0, n0)
    npass = jnp.where(c == 0, n0, BT - n0)

    def pass_body(p, _):
        bt = start + p
        _zero_slice(zb_v, acc, row0)
        plsc.subcore_barrier()
        ysrc = y_hbm.at[bt]

        for b in range(_NBUF):
            pltpu.async_copy(ysrc.at[src_v.at[b]], rows_v.at[b], gsems[b])

        def group(g, _):
            base = g * _NBUF
            for b in range(_NBUF):
                ch = base + b
                pltpu.make_async_copy(ysrc.at[src_v.at[ch]], rows_v.at[b],
                                      gsems[b]).wait()
                pltpu.sync_copy(rows_v.at[b], acc.at[dst_v.at[ch]], add=True)
                nxt = ch + _NBUF

                @pl.when(nxt < nch)
                def _():
                    pltpu.async_copy(ysrc.at[src_v.at[nxt]], rows_v.at[b],
                                     gsems[b])
            return 0

        lax.fori_loop(0, nch // _NBUF, group, 0)
        plsc.subcore_barrier()
        _write_slice(s, N, acc, out_hbm.at[bt])
        return 0

    lax.fori_loop(0, npass, pass_body, 0)


@functools.lru_cache(maxsize=None)
def _make_sc_aggregate(bt, n, w, nch, make_cnt):
    mesh = plsc.VectorSubcoreMesh(core_axis_name="c", subcore_axis_name="s")
    n_pad = _RPT * _NTILES
    out_type = [jax.ShapeDtypeStruct((bt, n, w), jnp.float32)]
    scratch = [
        pltpu.VMEM((nch, _CH), jnp.int32),
        pltpu.VMEM((nch, _CH), jnp.int32),
        pltpu.VMEM((_NBUF, _CH, w), jnp.float32),
        pltpu.VMEM((80, w), jnp.float32),
    ]
    if make_cnt:
        out_type = out_type + [jax.ShapeDtypeStruct((n, w), jnp.float32)]
        scratch = scratch + [pltpu.VMEM((_CH, w), jnp.float32)]
    scratch = scratch + [pltpu.VMEM_SHARED((n_pad, w), jnp.float32)]
    scratch = scratch + [pltpu.SemaphoreType.DMA] * _NBUF
    return functools.partial(
        pl.kernel,
        out_type=out_type,
        scratch_types=scratch,
        mesh=mesh,
        compiler_params=pltpu.CompilerParams(use_tc_tiling_on_sc=False),
    )(functools.partial(_agg_body, make_cnt))


def _sc_aggregate(y, src3, dst3, make_cnt=False):
    BT, N, W = y.shape
    return _make_sc_aggregate(BT, N, W, src3.shape[1], make_cnt)(y, src3, dst3)


def kernel(x_seq, edge_index, W_neigh0, b_neigh0, W_root0, ln_g0, ln_b0,
           W_neigh1, b_neigh1, W_root1, ln_g1, ln_b1,
           gru_Wih, gru_Whh, gru_bih, gru_bhh, head_W, head_b):
    B, T, N, F = x_seq.shape
    E = edge_index.shape[1]
    Hg = W_neigh0.shape[1]
    Ht = gru_Whh.shape[1]
    BT = B * T
    M = BT * N

    ept = E // _NTILES
    src3 = edge_index[0].astype(jnp.int32).reshape(_NTILES, ept // _CH, _CH)
    dst3 = edge_index[1].astype(jnp.int32).reshape(_NTILES, ept // _CH, _CH)

    x = x_seq.reshape(M, F)
    P0, R0 = _project(x, W_neigh0, W_root0, mb=2000)

    P0v = P0.reshape(BT, N, Hg)
    R0v = R0.reshape(BT, N, Hg)
    H2 = BT // 2
    ln0 = (b_neigh0.reshape(1, Hg), ln_g0.reshape(1, Hg), ln_b0.reshape(1, Hg))
    ln1 = (b_neigh1.reshape(1, Hg), ln_g1.reshape(1, Hg), ln_b1.reshape(1, Hg))

    agg0a, cntb = _sc_aggregate(P0v[:H2], src3, dst3, make_cnt=True)
    agg0b = _sc_aggregate(P0v[H2:], src3, dst3)[0]
    P1a, R1a = _fused_ln_proj(agg0a, R0v[:H2], cntb, *ln0,
                              W_neigh1, W_root1, nb=2000)
    P1b, R1b = _fused_ln_proj(agg0b, R0v[H2:], cntb, *ln0,
                              W_neigh1, W_root1, nb=2000)

    agg1a = _sc_aggregate(P1a, src3, dst3)[0]
    agg1b = _sc_aggregate(P1b, src3, dst3)[0]
    hsum_a = _ln_mean(agg1a, R1a, cntb, *ln1, nb=2000)
    hsum_b = _ln_mean(agg1b, R1b, cntb, *ln1, nb=2000)
    hsum = jnp.concatenate([hsum_a, hsum_b], axis=0)

    y = _gru_head(hsum.reshape(B, T, Hg), gru_Wih.T, gru_Whh.T,
                  gru_bih.reshape(1, 3 * Ht), gru_bhh.reshape(1, 3 * Ht),
                  head_W, head_b.reshape(1, 1), n_nodes=N)
    return y[:, 0]


# speedup vs baseline: 25.2401x; 1.0583x over previous
import functools

import jax
import jax.numpy as jnp
from jax import lax
from jax.experimental import pallas as pl
from jax.experimental.pallas import tpu as pltpu
from jax.experimental.pallas import tpu_sc as plsc


def _proj_body(x_ref, wn_ref, wr_ref, p_ref, r_ref):
    x = x_ref[...]
    p_ref[...] = jnp.dot(x, wn_ref[...], preferred_element_type=jnp.float32)
    r_ref[...] = jnp.dot(x, wr_ref[...], preferred_element_type=jnp.float32)


def _ln_from_parts(agg, r, cnt, bn, g, b):
    inv = 1.0 / jnp.maximum(cnt[0][:, 0:1] + cnt[1][:, 0:1], 1.0)
    z = agg * inv + bn + r
    mu = jnp.mean(z, axis=-1, keepdims=True)
    var = jnp.mean((z - mu) ** 2, axis=-1, keepdims=True)
    h = (z - mu) * lax.rsqrt(var + 1e-5) * g + b
    return jnp.maximum(h, 0.0)


def _fused_ln_proj_body(agg_ref, r_ref, cnt_ref, bn_ref, g_ref, b_ref,
                        wn_ref, wr_ref, p1_ref, r1_ref):
    h = _ln_from_parts(agg_ref[0], r_ref[0], cnt_ref[...],
                       bn_ref[...], g_ref[...], b_ref[...])
    p1_ref[0] = jnp.dot(h, wn_ref[...], preferred_element_type=jnp.float32)
    r1_ref[0] = jnp.dot(h, wr_ref[...], preferred_element_type=jnp.float32)


def _ln_mean_body(agg_ref, r_ref, cnt_ref, bn_ref, g_ref, b_ref, hsum_ref):
    nb = pl.program_id(1)
    h = _ln_from_parts(agg_ref[0], r_ref[0], cnt_ref[...],
                       bn_ref[...], g_ref[...], b_ref[...])

    @pl.when(nb == 0)
    def _():
        hsum_ref[...] = jnp.zeros_like(hsum_ref)

    s = jnp.sum(h, axis=0)
    hsum_ref[...] += jnp.broadcast_to(s[None, None, :], hsum_ref.shape)


def _gru_body(hseq_ref, wih_ref, whh_ref, bih_ref, bhh_ref, hw_ref, hb_ref,
              y_ref, *, n_nodes):
    B, T, _ = hseq_ref.shape
    Ht = whh_ref.shape[0]
    h = jnp.zeros((B, Ht), jnp.float32)
    scale = 1.0 / float(n_nodes)
    for t in range(T):
        ht = hseq_ref[:, t, :] * scale
        gi = jnp.dot(ht, wih_ref[...], preferred_element_type=jnp.float32) + bih_ref[...]
        gh = jnp.dot(h, whh_ref[...], preferred_element_type=jnp.float32) + bhh_ref[...]
        ir, iz, inn = gi[:, :Ht], gi[:, Ht:2 * Ht], gi[:, 2 * Ht:]
        hr, hz, hn = gh[:, :Ht], gh[:, Ht:2 * Ht], gh[:, 2 * Ht:]
        r = jax.nn.sigmoid(ir + hr)
        z = jax.nn.sigmoid(iz + hz)
        n = jnp.tanh(inn + r * hn)
        h = (1.0 - z) * n + z * h
    y_ref[...] = jnp.dot(h, hw_ref[...], preferred_element_type=jnp.float32) + hb_ref[...]


def _project(x, wn, wr, mb):
    M, F = x.shape
    Hg = wn.shape[1]
    return pl.pallas_call(
        _proj_body,
        grid=(M // mb,),
        in_specs=[
            pl.BlockSpec((mb, F), lambda i: (i, 0)),
            pl.BlockSpec((F, Hg), lambda i: (0, 0)),
            pl.BlockSpec((F, Hg), lambda i: (0, 0)),
        ],
        out_specs=[
            pl.BlockSpec((mb, Hg), lambda i: (i, 0)),
            pl.BlockSpec((mb, Hg), lambda i: (i, 0)),
        ],
        out_shape=[
            jax.ShapeDtypeStruct((M, Hg), jnp.float32),
            jax.ShapeDtypeStruct((M, Hg), jnp.float32),
        ],
    )(x, wn, wr)


def _fused_ln_proj(agg, r, cntb, bn, g, b, wn, wr, nb):
    BT, N, Hg = agg.shape
    grid = (BT, N // nb)
    row = pl.BlockSpec((1, nb, Hg), lambda i, j: (i, j, 0))
    vec = pl.BlockSpec((1, Hg), lambda i, j: (0, 0))
    mat = pl.BlockSpec((Hg, Hg), lambda i, j: (0, 0))
    cnt = pl.BlockSpec((2, nb, cntb.shape[2]), lambda i, j: (0, j, 0))
    return pl.pallas_call(
        _fused_ln_proj_body,
        grid=grid,
        in_specs=[row, row, cnt, vec, vec, vec, mat, mat],
        out_specs=[row, row],
        out_shape=[
            jax.ShapeDtypeStruct((BT, N, Hg), jnp.float32),
            jax.ShapeDtypeStruct((BT, N, Hg), jnp.float32),
        ],
    )(agg, r, cntb, bn, g, b, wn, wr)


def _ln_mean(agg, r, cntb, bn, g, b, nb):
    BT, N, Hg = agg.shape
    grid = (BT, N // nb)
    row = pl.BlockSpec((1, nb, Hg), lambda i, j: (i, j, 0))
    vec = pl.BlockSpec((1, Hg), lambda i, j: (0, 0))
    cnt = pl.BlockSpec((2, nb, cntb.shape[2]), lambda i, j: (0, j, 0))
    return pl.pallas_call(
        _ln_mean_body,
        grid=grid,
        in_specs=[row, row, cnt, vec, vec, vec],
        out_specs=pl.BlockSpec((1, 8, Hg), lambda i, j: (i, 0, 0)),
        out_shape=jax.ShapeDtypeStruct((BT, 8, Hg), jnp.float32),
    )(agg, r, cntb, bn, g, b)[:, 0, :]


def _gru_head(hseq, wih_t, whh_t, bih, bhh, head_w, head_b, n_nodes):
    B, T, Hg = hseq.shape
    Ht = whh_t.shape[0]
    full = lambda s: pl.BlockSpec(s, lambda: tuple(0 for _ in s))
    return pl.pallas_call(
        functools.partial(_gru_body, n_nodes=n_nodes),
        in_specs=[full((B, T, Hg)), full((Hg, 3 * Ht)), full((Ht, 3 * Ht)),
                  full((1, 3 * Ht)), full((1, 3 * Ht)), full((Ht, 1)),
                  full((1, 1))],
        out_specs=full((B, 1)),
        out_shape=jax.ShapeDtypeStruct((B, 1), jnp.float32),
    )(hseq, wih_t, whh_t, bih, bhh, head_w, head_b)


_NTILES = 16
_CH = 100
_RPT = 640


def _fill(ref, value):
    rows, cols = ref.shape

    def body(i, _):
        v = jnp.full((16,), value, jnp.float32)
        for j in range(cols // 16):
            ref[i, pl.ds(j * 16, 16)] = v
        return 0

    lax.fori_loop(0, rows, body, 0)


def _tile_slices(s, n):
    row0 = pl.multiple_of(s * _RPT, 8)
    last = n - _RPT * (_NTILES - 1)
    return row0, last


def _zero_slice(zb_v, dst_ref, row0):
    zrows = zb_v.shape[0]
    for j in range(_RPT // zrows):
        pltpu.sync_copy(zb_v, dst_ref.at[pl.ds(row0 + j * zrows, zrows)])


def _write_slice(s, n, acc_ref, hbm_ref):
    row0, last = _tile_slices(s, n)

    @pl.when(s < _NTILES - 1)
    def _():
        pltpu.sync_copy(acc_ref.at[pl.ds(row0, _RPT)],
                        hbm_ref.at[pl.ds(row0, _RPT)])

    @pl.when(s == _NTILES - 1)
    def _():
        base = _RPT * (_NTILES - 1)
        pltpu.sync_copy(acc_ref.at[pl.ds(base, last)],
                        hbm_ref.at[pl.ds(base, last)])


_NBUF = 5


def _agg_body(make_cnt, y_hbm, src3_hbm, dst3_hbm, out_hbm, *rest):
    if make_cnt:
        cnt_hbm, src_v, dst_v, rows_v, zb_v, ones_v, acc = rest[:7]
        gsems = rest[7:]
    else:
        src_v, dst_v, rows_v, zb_v, acc = rest[:5]
        gsems = rest[5:]
    BT, N, W = y_hbm.shape
    nch = src_v.shape[0]
    c = lax.axis_index("c")
    s = lax.axis_index("s")
    row0, _ = _tile_slices(s, N)

    pltpu.sync_copy(src3_hbm.at[s], src_v)
    pltpu.sync_copy(dst3_hbm.at[s], dst_v)
    _fill(zb_v, 0.0)

    if make_cnt:
        _fill(ones_v, 1.0)
        _zero_slice(zb_v, acc, row0)
        plsc.subcore_barrier()

        def cbody(ch, _):
            pltpu.sync_copy(ones_v, acc.at[dst_v.at[ch]], add=True)
            return 0

        half = nch // 2
        lax.fori_loop(c * half, c * half + half, cbody, 0)
        plsc.subcore_barrier()
        _write_slice(s, N, acc, cnt_hbm.at[c])

    n0 = BT // 2
    start = jnp.where(c == 0, 0, n0)
    npass = jnp.where(c == 0, n0, BT - n0)

    def pass_body(p, _):
        bt = start + p
        _zero_slice(zb_v, acc, row0)
        plsc.subcore_barrier()
        ysrc = y_hbm.at[bt]

        for b in range(_NBUF):
            pltpu.async_copy(ysrc.at[src_v.at[b]], rows_v.at[b], gsems[b])

        def group(g, _):
            base = g * _NBUF
            for b in range(_NBUF):
                ch = base + b
                pltpu.make_async_copy(ysrc.at[src_v.at[ch]], rows_v.at[b],
                                      gsems[b]).wait()
                pltpu.sync_copy(rows_v.at[b], acc.at[dst_v.at[ch]], add=True)
                nxt = ch + _NBUF

                @pl.when(nxt < nch)
                def _():
                    pltpu.async_copy(ysrc.at[src_v.at[nxt]], rows_v.at[b],
                                     gsems[b])
            return 0

        lax.fori_loop(0, nch // _NBUF, group, 0)
        plsc.subcore_barrier()
        _write_slice(s, N, acc, out_hbm.at[bt])
        return 0

    lax.fori_loop(0, npass, pass_body, 0)


@functools.lru_cache(maxsize=None)
def _make_sc_aggregate(bt, n, w, nch, make_cnt):
    mesh = plsc.VectorSubcoreMesh(core_axis_name="c", subcore_axis_name="s")
    n_pad = _RPT * _NTILES
    out_type = [jax.ShapeDtypeStruct((bt, n, w), jnp.float32)]
    scratch = [
        pltpu.VMEM((nch, _CH), jnp.int32),
        pltpu.VMEM((nch, _CH), jnp.int32),
        pltpu.VMEM((_NBUF, _CH, w), jnp.float32),
        pltpu.VMEM((80, w), jnp.float32),
    ]
    if make_cnt:
        out_type = out_type + [jax.ShapeDtypeStruct((2, n, w), jnp.float32)]
        scratch = scratch + [pltpu.VMEM((_CH, w), jnp.float32)]
    scratch = scratch + [pltpu.VMEM_SHARED((n_pad, w), jnp.float32)]
    scratch = scratch + [pltpu.SemaphoreType.DMA] * _NBUF
    return functools.partial(
        pl.kernel,
        out_type=out_type,
        scratch_types=scratch,
        mesh=mesh,
        compiler_params=pltpu.CompilerParams(use_tc_tiling_on_sc=False),
    )(functools.partial(_agg_body, make_cnt))


def _sc_aggregate(y, src3, dst3, make_cnt=False):
    BT, N, W = y.shape
    return _make_sc_aggregate(BT, N, W, src3.shape[1], make_cnt)(y, src3, dst3)


def kernel(x_seq, edge_index, W_neigh0, b_neigh0, W_root0, ln_g0, ln_b0,
           W_neigh1, b_neigh1, W_root1, ln_g1, ln_b1,
           gru_Wih, gru_Whh, gru_bih, gru_bhh, head_W, head_b):
    B, T, N, F = x_seq.shape
    E = edge_index.shape[1]
    Hg = W_neigh0.shape[1]
    Ht = gru_Whh.shape[1]
    BT = B * T
    M = BT * N

    ept = E // _NTILES
    src3 = edge_index[0].astype(jnp.int32).reshape(_NTILES, ept // _CH, _CH)
    dst3 = edge_index[1].astype(jnp.int32).reshape(_NTILES, ept // _CH, _CH)

    x = x_seq.reshape(M, F)
    H2 = BT // 2
    P0a, R0a = _project(x[:M // 2], W_neigh0, W_root0, mb=2000)
    P0b, R0b = _project(x[M // 2:], W_neigh0, W_root0, mb=2000)
    ln0 = (b_neigh0.reshape(1, Hg), ln_g0.reshape(1, Hg), ln_b0.reshape(1, Hg))
    ln1 = (b_neigh1.reshape(1, Hg), ln_g1.reshape(1, Hg), ln_b1.reshape(1, Hg))

    agg0a, cntb = _sc_aggregate(P0a.reshape(H2, N, Hg), src3, dst3,
                                make_cnt=True)
    agg0b = _sc_aggregate(P0b.reshape(H2, N, Hg), src3, dst3)[0]
    P1a, R1a = _fused_ln_proj(agg0a, R0a.reshape(H2, N, Hg), cntb, *ln0,
                              W_neigh1, W_root1, nb=2000)
    P1b, R1b = _fused_ln_proj(agg0b, R0b.reshape(H2, N, Hg), cntb, *ln0,
                              W_neigh1, W_root1, nb=2000)

    agg1a = _sc_aggregate(P1a, src3, dst3)[0]
    agg1b = _sc_aggregate(P1b, src3, dst3)[0]
    hsum_a = _ln_mean(agg1a, R1a, cntb, *ln1, nb=2000)
    hsum_b = _ln_mean(agg1b, R1b, cntb, *ln1, nb=2000)
    hsum = jnp.concatenate([hsum_a, hsum_b], axis=0)

    y = _gru_head(hsum.reshape(B, T, Hg), gru_Wih.T, gru_Whh.T,
                  gru_bih.reshape(1, 3 * Ht), gru_bhh.reshape(1, 3 * Ht),
                  head_W, head_b.reshape(1, 1), n_nodes=N)
    return y[:, 0]


# speedup vs baseline: 25.6956x; 1.0180x over previous
import functools

import jax
import jax.numpy as jnp
from jax import lax
from jax.experimental import pallas as pl
from jax.experimental.pallas import tpu as pltpu
from jax.experimental.pallas import tpu_sc as plsc


def _proj_body(x_ref, wn_ref, wr_ref, p_ref, r_ref):
    x = x_ref[...]
    p_ref[...] = jnp.dot(x, wn_ref[...], preferred_element_type=jnp.float32)
    r_ref[...] = jnp.dot(x, wr_ref[...], preferred_element_type=jnp.float32)


def _ln_from_parts(agg, r, cnt, bn, g, b):
    inv = 1.0 / jnp.maximum(cnt[0][:, 0:1] + cnt[1][:, 0:1], 1.0)
    z = agg * inv + bn + r
    mu = jnp.mean(z, axis=-1, keepdims=True)
    var = jnp.mean((z - mu) ** 2, axis=-1, keepdims=True)
    h = (z - mu) * lax.rsqrt(var + 1e-5) * g + b
    return jnp.maximum(h, 0.0)


def _fused_ln_proj_body(agg_ref, r_ref, cnt_ref, bn_ref, g_ref, b_ref,
                        wn_ref, wr_ref, p1_ref, r1_ref):
    h = _ln_from_parts(agg_ref[0], r_ref[0], cnt_ref[...],
                       bn_ref[...], g_ref[...], b_ref[...])
    p1_ref[0] = jnp.dot(h, wn_ref[...], preferred_element_type=jnp.float32)
    r1_ref[0] = jnp.dot(h, wr_ref[...], preferred_element_type=jnp.float32)


def _ln_mean_body(agg_ref, r_ref, cnt_ref, bn_ref, g_ref, b_ref, hsum_ref):
    nb = pl.program_id(1)
    h = _ln_from_parts(agg_ref[0], r_ref[0], cnt_ref[...],
                       bn_ref[...], g_ref[...], b_ref[...])

    @pl.when(nb == 0)
    def _():
        hsum_ref[...] = jnp.zeros_like(hsum_ref)

    s = jnp.sum(h, axis=0)
    hsum_ref[...] += jnp.broadcast_to(s[None, None, :], hsum_ref.shape)


def _gru_body(hseq_ref, wih_ref, whh_ref, bih_ref, bhh_ref, hw_ref, hb_ref,
              y_ref, *, n_nodes):
    B, T, _ = hseq_ref.shape
    Ht = whh_ref.shape[0]
    h = jnp.zeros((B, Ht), jnp.float32)
    scale = 1.0 / float(n_nodes)
    for t in range(T):
        ht = hseq_ref[:, t, :] * scale
        gi = jnp.dot(ht, wih_ref[...], preferred_element_type=jnp.float32) + bih_ref[...]
        gh = jnp.dot(h, whh_ref[...], preferred_element_type=jnp.float32) + bhh_ref[...]
        ir, iz, inn = gi[:, :Ht], gi[:, Ht:2 * Ht], gi[:, 2 * Ht:]
        hr, hz, hn = gh[:, :Ht], gh[:, Ht:2 * Ht], gh[:, 2 * Ht:]
        r = jax.nn.sigmoid(ir + hr)
        z = jax.nn.sigmoid(iz + hz)
        n = jnp.tanh(inn + r * hn)
        h = (1.0 - z) * n + z * h
    y_ref[...] = jnp.dot(h, hw_ref[...], preferred_element_type=jnp.float32) + hb_ref[...]


def _project(x, wn, wr, mb):
    M, F = x.shape
    Hg = wn.shape[1]
    return pl.pallas_call(
        _proj_body,
        grid=(M // mb,),
        in_specs=[
            pl.BlockSpec((mb, F), lambda i: (i, 0)),
            pl.BlockSpec((F, Hg), lambda i: (0, 0)),
            pl.BlockSpec((F, Hg), lambda i: (0, 0)),
        ],
        out_specs=[
            pl.BlockSpec((mb, Hg), lambda i: (i, 0)),
            pl.BlockSpec((mb, Hg), lambda i: (i, 0)),
        ],
        out_shape=[
            jax.ShapeDtypeStruct((M, Hg), jnp.float32),
            jax.ShapeDtypeStruct((M, Hg), jnp.float32),
        ],
    )(x, wn, wr)


def _fused_ln_proj(agg, r, cntb, bn, g, b, wn, wr, nb):
    BT, N, Hg = agg.shape
    grid = (BT, N // nb)
    row = pl.BlockSpec((1, nb, Hg), lambda i, j: (i, j, 0))
    vec = pl.BlockSpec((1, Hg), lambda i, j: (0, 0))
    mat = pl.BlockSpec((Hg, Hg), lambda i, j: (0, 0))
    cnt = pl.BlockSpec((2, nb, cntb.shape[2]), lambda i, j: (0, j, 0))
    return pl.pallas_call(
        _fused_ln_proj_body,
        grid=grid,
        in_specs=[row, row, cnt, vec, vec, vec, mat, mat],
        out_specs=[row, row],
        out_shape=[
            jax.ShapeDtypeStruct((BT, N, Hg), jnp.float32),
            jax.ShapeDtypeStruct((BT, N, Hg), jnp.float32),
        ],
    )(agg, r, cntb, bn, g, b, wn, wr)


def _ln_mean(agg, r, cntb, bn, g, b, nb):
    BT, N, Hg = agg.shape
    grid = (BT, N // nb)
    row = pl.BlockSpec((1, nb, Hg), lambda i, j: (i, j, 0))
    vec = pl.BlockSpec((1, Hg), lambda i, j: (0, 0))
    cnt = pl.BlockSpec((2, nb, cntb.shape[2]), lambda i, j: (0, j, 0))
    return pl.pallas_call(
        _ln_mean_body,
        grid=grid,
        in_specs=[row, row, cnt, vec, vec, vec],
        out_specs=pl.BlockSpec((1, 8, Hg), lambda i, j: (i, 0, 0)),
        out_shape=jax.ShapeDtypeStruct((BT, 8, Hg), jnp.float32),
    )(agg, r, cntb, bn, g, b)[:, 0, :]


def _gru_head(hseq, wih_t, whh_t, bih, bhh, head_w, head_b, n_nodes):
    B, T, Hg = hseq.shape
    Ht = whh_t.shape[0]
    full = lambda s: pl.BlockSpec(s, lambda: tuple(0 for _ in s))
    return pl.pallas_call(
        functools.partial(_gru_body, n_nodes=n_nodes),
        in_specs=[full((B, T, Hg)), full((Hg, 3 * Ht)), full((Ht, 3 * Ht)),
                  full((1, 3 * Ht)), full((1, 3 * Ht)), full((Ht, 1)),
                  full((1, 1))],
        out_specs=full((B, 1)),
        out_shape=jax.ShapeDtypeStruct((B, 1), jnp.float32),
    )(hseq, wih_t, whh_t, bih, bhh, head_w, head_b)


_NTILES = 16
_CH = 100
_RPT = 640


def _fill(ref, value):
    rows, cols = ref.shape

    def body(i, _):
        v = jnp.full((16,), value, jnp.float32)
        for j in range(cols // 16):
            ref[i, pl.ds(j * 16, 16)] = v
        return 0

    lax.fori_loop(0, rows, body, 0)


def _tile_slices(s, n):
    row0 = pl.multiple_of(s * _RPT, 8)
    last = n - _RPT * (_NTILES - 1)
    return row0, last


def _zero_slice(zb_v, dst_ref, row0):
    zrows = zb_v.shape[0]
    for j in range(_RPT // zrows):
        pltpu.sync_copy(zb_v, dst_ref.at[pl.ds(row0 + j * zrows, zrows)])


def _write_slice(s, n, acc_ref, hbm_ref):
    row0, last = _tile_slices(s, n)

    @pl.when(s < _NTILES - 1)
    def _():
        pltpu.sync_copy(acc_ref.at[pl.ds(row0, _RPT)],
                        hbm_ref.at[pl.ds(row0, _RPT)])

    @pl.when(s == _NTILES - 1)
    def _():
        base = _RPT * (_NTILES - 1)
        pltpu.sync_copy(acc_ref.at[pl.ds(base, last)],
                        hbm_ref.at[pl.ds(base, last)])


_NBUF = 5


def _agg_body(make_cnt, y_hbm, src3_hbm, dst3_hbm, out_hbm, *rest):
    if make_cnt:
        cnt_hbm, src_v, dst_v, rows_v, zb_v, ones_v, acc = rest[:7]
        gsems = rest[7:]
    else:
        src_v, dst_v, rows_v, zb_v, acc = rest[:5]
        gsems = rest[5:]
    BT, N, W = y_hbm.shape
    nch = src_v.shape[0]
    c = lax.axis_index("c")
    s = lax.axis_index("s")
    row0, _ = _tile_slices(s, N)

    pltpu.sync_copy(src3_hbm.at[s], src_v)
    pltpu.sync_copy(dst3_hbm.at[s], dst_v)
    _fill(zb_v, 0.0)

    if make_cnt:
        _fill(ones_v, 1.0)
        _zero_slice(zb_v, acc, row0)
        plsc.subcore_barrier()

        def cbody(ch, _):
            pltpu.sync_copy(ones_v, acc.at[dst_v.at[ch]], add=True)
            return 0

        half = nch // 2
        lax.fori_loop(c * half, c * half + half, cbody, 0)
        plsc.subcore_barrier()
        _write_slice(s, N, acc, cnt_hbm.at[c])

    n0 = BT // 2
    start = jnp.where(c == 0, 0, n0)
    npass = jnp.where(c == 0, n0, BT - n0)

    def pass_body(p, _):
        bt = start + p
        _zero_slice(zb_v, acc, row0)
        plsc.subcore_barrier()
        ysrc = y_hbm.at[bt]

        for b in range(_NBUF):
            pltpu.async_copy(ysrc.at[src_v.at[b]], rows_v.at[b], gsems[b])

        def group(g, _):
            base = g * _NBUF
            for b in range(_NBUF):
                ch = base + b
                pltpu.make_async_copy(ysrc.at[src_v.at[ch]], rows_v.at[b],
                                      gsems[b]).wait()
                pltpu.sync_copy(rows_v.at[b], acc.at[dst_v.at[ch]], add=True)
                nxt = ch + _NBUF

                @pl.when(nxt < nch)
                def _():
                    pltpu.async_copy(ysrc.at[src_v.at[nxt]], rows_v.at[b],
                                     gsems[b])
            return 0

        lax.fori_loop(0, nch // _NBUF, group, 0)
        plsc.subcore_barrier()
        _write_slice(s, N, acc, out_hbm.at[bt])
        return 0

    lax.fori_loop(0, npass, pass_body, 0)


@functools.lru_cache(maxsize=None)
def _make_sc_aggregate(bt, n, w, nch, make_cnt):
    mesh = plsc.VectorSubcoreMesh(core_axis_name="c", subcore_axis_name="s")
    n_pad = _RPT * _NTILES
    out_type = [jax.ShapeDtypeStruct((bt, n, w), jnp.float32)]
    scratch = [
        pltpu.VMEM((nch, _CH), jnp.int32),
        pltpu.VMEM((nch, _CH), jnp.int32),
        pltpu.VMEM((_NBUF, _CH, w), jnp.float32),
        pltpu.VMEM((80, w), jnp.float32),
    ]
    if make_cnt:
        out_type = out_type + [jax.ShapeDtypeStruct((2, n, w), jnp.float32)]
        scratch = scratch + [pltpu.VMEM((_CH, w), jnp.float32)]
    scratch = scratch + [pltpu.VMEM_SHARED((n_pad, w), jnp.float32)]
    scratch = scratch + [pltpu.SemaphoreType.DMA] * _NBUF
    return functools.partial(
        pl.kernel,
        out_type=out_type,
        scratch_types=scratch,
        mesh=mesh,
        compiler_params=pltpu.CompilerParams(use_tc_tiling_on_sc=False),
    )(functools.partial(_agg_body, make_cnt))


def _sc_aggregate(y, src3, dst3, make_cnt=False):
    BT, N, W = y.shape
    return _make_sc_aggregate(BT, N, W, src3.shape[1], make_cnt)(y, src3, dst3)


def kernel(x_seq, edge_index, W_neigh0, b_neigh0, W_root0, ln_g0, ln_b0,
           W_neigh1, b_neigh1, W_root1, ln_g1, ln_b1,
           gru_Wih, gru_Whh, gru_bih, gru_bhh, head_W, head_b):
    B, T, N, F = x_seq.shape
    E = edge_index.shape[1]
    Hg = W_neigh0.shape[1]
    Ht = gru_Whh.shape[1]
    BT = B * T
    M = BT * N

    ept = E // _NTILES
    src3 = edge_index[0].astype(jnp.int32).reshape(_NTILES, ept // _CH, _CH)
    dst3 = edge_index[1].astype(jnp.int32).reshape(_NTILES, ept // _CH, _CH)

    x = x_seq.reshape(M, F)
    nparts = 4
    pb = BT // nparts
    mp = M // nparts
    ln0 = (b_neigh0.reshape(1, Hg), ln_g0.reshape(1, Hg), ln_b0.reshape(1, Hg))
    ln1 = (b_neigh1.reshape(1, Hg), ln_g1.reshape(1, Hg), ln_b1.reshape(1, Hg))

    proj = [_project(x[i * mp:(i + 1) * mp], W_neigh0, W_root0, mb=2000)
            for i in range(nparts)]

    cntb = None
    agg0 = []
    for i, (p0, _) in enumerate(proj):
        if i == 0:
            a, cntb = _sc_aggregate(p0.reshape(pb, N, Hg), src3, dst3,
                                    make_cnt=True)
        else:
            a = _sc_aggregate(p0.reshape(pb, N, Hg), src3, dst3)[0]
        agg0.append(a)

    pr1 = [_fused_ln_proj(agg0[i], proj[i][1].reshape(pb, N, Hg), cntb, *ln0,
                          W_neigh1, W_root1, nb=2000)
           for i in range(nparts)]

    agg1 = [_sc_aggregate(pr1[i][0], src3, dst3)[0] for i in range(nparts)]
    hsums = [_ln_mean(agg1[i], pr1[i][1], cntb, *ln1, nb=2000)
             for i in range(nparts)]
    hsum = jnp.concatenate(hsums, axis=0)

    y = _gru_head(hsum.reshape(B, T, Hg), gru_Wih.T, gru_Whh.T,
                  gru_bih.reshape(1, 3 * Ht), gru_bhh.reshape(1, 3 * Ht),
                  head_W, head_b.reshape(1, 1), n_nodes=N)
    return y[:, 0]
